# baseline (speedup 1.0000x reference)
import jax
import jax.numpy as jnp
from jax import lax
from jax.experimental import pallas as pl
from jax.experimental.pallas import tpu as pltpu
from jax.experimental.pallas import tpu_sc as plsc

NU = 50000
NI = 20000
NENT = 50000
NREL = 17
EMB = 64
HOPS = 2
NEDGE = 800000
BATCH = 4096
NEG = 16
MARGIN = 0.8
DECAY = 1e-4
ANGLE_W = 0.5
ANGLE_DROP = 0.5

NC, NS, L = 2, 16, 16
CH = 128
SB = 56
NSB = 7
EPT = CH * SB * NSB
EPAD = EPT * NS

NR = EPAD // CH
ACC_E = 50176
RPT_E = ACC_E // NS
ACC_I = 20480
RPT_I = ACC_I // NS


def _mesh():
    return plsc.VectorSubcoreMesh(core_axis_name="c", subcore_axis_name="s")


def _make_segsum(tbl_rows4, accr, rpt, mode, off):
    scratch = [
        pltpu.VMEM((SB, 3, CH), jnp.int32),
        pltpu.VMEM((2, CH, L), jnp.float32),
        pltpu.VMEM((112, L), jnp.float32),
        pltpu.VMEM_SHARED((accr, L), jnp.float32),
        pltpu.SemaphoreType.DMA,
        pltpu.SemaphoreType.DMA,
    ]
    if mode == "rel":
        scratch.insert(1, pltpu.VMEM((4 * (NREL - 1), L), jnp.float32))

    def body(tbl, edges, *rest):
        if mode == "rel":
            rel, out, esb, relv, rows, zbuf, acc, sem0, sem1 = rest
        else:
            out, esb, rows, zbuf, acc, sem0, sem1 = rest
            relv = None
        c = lax.axis_index("c")
        s = lax.axis_index("s")
        zv = jnp.zeros((L,), jnp.float32)
        for zi in range(112):
            zbuf[zi, pl.ds(0, L)] = zv
        if mode == "rel":
            pltpu.sync_copy(rel, relv)

        row0 = s * (SB * NSB)
        sems = [sem0, sem1]

        for p in range(2):
            q = c + 2 * p

            def zloop(k, _):
                pltpu.sync_copy(zbuf, acc.at[pl.ds(s * rpt + k * 112, 112), :])
                return 0
            lax.fori_loop(0, rpt // 112, zloop, 0)
            plsc.subcore_barrier()

            lanes = lax.iota(jnp.int32, L)

            def chunk(j, b):
                pltpu.make_async_copy(tbl.at[esb.at[j, 0]], rows.at[b],
                                      sems[b]).wait()
                fb = jnp.full((L,), b, jnp.int32)

                def grp(g, _):
                    i0 = g * L
                    ie = lanes + i0
                    if mode == "rel":
                        rr = 4 * esb[j, 2, pl.ds(i0, L)] + (q - 4)
                    else:
                        vv = plsc.bitcast(esb[j, 2, pl.ds(i0, L)], jnp.float32)
                    for d in range(L):
                        fd = jnp.full((L,), d, jnp.int32)
                        r_ = plsc.load_gather(rows, [fb, ie, fd])
                        if mode == "rel":
                            m_ = plsc.load_gather(relv, [rr, fd])
                        else:
                            m_ = vv
                        plsc.store_scatter(rows, [fb, ie, fd], r_ * m_)
                    return 0
                lax.fori_loop(0, CH // L, grp, 0)
                pltpu.sync_copy(rows.at[b], acc.at[esb.at[j, 1]], add=True)

            def super_chunk(sb, _):
                r0 = row0 + sb * SB
                pltpu.sync_copy(edges.at[pl.ds(r0, SB), :, :], esb)

                def txf(r, _):
                    for k in range(CH // L):
                        v = esb[r, 0, pl.ds(k * L, L)]
                        esb[r, 0, pl.ds(k * L, L)] = v * 4 + (q + 4 * off)
                    return 0
                lax.fori_loop(0, SB, txf, 0)

                pltpu.async_copy(tbl.at[esb.at[0, 0]], rows.at[0], sem0)
                pltpu.async_copy(tbl.at[esb.at[1, 0]], rows.at[1], sem1)

                def loop_body(m, _):
                    j0 = 2 * m
                    chunk(j0, 0)

                    @pl.when(j0 + 2 < SB)
                    def _():
                        pltpu.async_copy(tbl.at[esb.at[j0 + 2, 0]], rows.at[0],
                                         sem0)

                    chunk(j0 + 1, 1)

                    @pl.when(j0 + 3 < SB)
                    def _():
                        pltpu.async_copy(tbl.at[esb.at[j0 + 3, 0]], rows.at[1],
                                         sem1)
                    return 0

                lax.fori_loop(0, SB // 2, loop_body, 0)
                return 0

            lax.fori_loop(0, NSB, super_chunk, 0)
            plsc.subcore_barrier()

            pltpu.sync_copy(acc.at[pl.ds(s * rpt, rpt), :],
                            out.at[pl.ds(s * rpt, rpt), q, :])
            plsc.subcore_barrier()

    del tbl_rows4
    return pl.kernel(
        body,
        out_type=jax.ShapeDtypeStruct((accr, 4, L), jnp.float32),
        mesh=_mesh(),
        scratch_types=scratch,
        compiler_params=pltpu.CompilerParams(use_tc_tiling_on_sc=False,
                                             needs_layout_passes=False),
    )


def _norm_res(x, res):
    R = x.shape[0]
    blk = 512

    def body(x_ref, r_ref, o_ref):
        xv = x_ref[...]
        nrm = jnp.sqrt(jnp.sum(xv * xv, axis=1, keepdims=True))
        o_ref[...] = r_ref[...] + xv / (nrm + 1e-8)

    return pl.pallas_call(
        body,
        out_shape=jax.ShapeDtypeStruct((R, EMB), jnp.float32),
        grid=(R // blk,),
        in_specs=[pl.BlockSpec((blk, EMB), lambda i: (i, 0)),
                  pl.BlockSpec((blk, EMB), lambda i: (i, 0))],
        out_specs=pl.BlockSpec((blk, EMB), lambda i: (i, 0)),
    )(x, res)


def _div_norm_res(x, deg8, res):
    R = x.shape[0]
    blk = 512

    def body(x_ref, d_ref, r_ref, e_ref, o_ref):
        xv = x_ref[...]
        d = jnp.maximum(d_ref[:, 0:1], 1.0)
        e_ref[...] = xv / d
        nrm = jnp.sqrt(jnp.sum(xv * xv, axis=1, keepdims=True))
        o_ref[...] = r_ref[...] + xv / (nrm + 1e-8)

    return pl.pallas_call(
        body,
        out_shape=(jax.ShapeDtypeStruct((R, EMB), jnp.float32),
                   jax.ShapeDtypeStruct((R, EMB), jnp.float32)),
        grid=(R // blk,),
        in_specs=[pl.BlockSpec((blk, EMB), lambda i: (i, 0)),
                  pl.BlockSpec((blk, 8), lambda i: (i, 0)),
                  pl.BlockSpec((blk, EMB), lambda i: (i, 0))],
        out_specs=(pl.BlockSpec((blk, EMB), lambda i: (i, 0)),
                   pl.BlockSpec((blk, EMB), lambda i: (i, 0))),
    )(x, deg8, res)


def _loss_partials(ue, pe, pc, ne2, nc2):
    blk = 512
    G = BATCH // blk

    def body(u_ref, p_ref, pc_ref, n_ref, nc_ref, o_ref, *o_rest):
        u_raw = u_ref[...]
        p_raw = p_ref[...]
        pc_raw = pc_ref[...]
        u = u_raw / (jnp.sqrt(jnp.sum(u_raw * u_raw, axis=1, keepdims=True)) + 1e-8)
        ps = p_raw + pc_raw
        p = ps / (jnp.sqrt(jnp.sum(ps * ps, axis=1, keepdims=True)) + 1e-8)
        pos_score = jnp.sum(u * p, axis=1, keepdims=True)
        hinge = jnp.maximum(1.0 - pos_score, 0.0)
        neg_acc = jnp.zeros((blk, 1), jnp.float32)
        sq_ne = jnp.zeros((), jnp.float32)
        sq_nc = jnp.zeros((), jnp.float32)
        for k in range(NEG):
            nk_raw = n_ref[:, k * EMB:(k + 1) * EMB]
            nck_raw = nc_ref[:, k * EMB:(k + 1) * EMB]
            sq_ne += jnp.sum(nk_raw * nk_raw)
            sq_nc += jnp.sum(nck_raw * nck_raw)
            nk = nk_raw + nck_raw
            nkn = nk / (jnp.sqrt(jnp.sum(nk * nk, axis=1, keepdims=True)) + 1e-8)
            ns = jnp.sum(u * nkn, axis=1, keepdims=True)
            neg_acc += jnp.maximum(ns - MARGIN, 0.0)
        tot = jnp.sum(hinge + neg_acc * (1.0 / NEG))
        cols = [tot,
                jnp.sum(u_raw * u_raw),
                jnp.sum(p_raw * p_raw),
                sq_ne,
                jnp.sum(pc_raw * pc_raw),
                sq_nc]
        orefs = [o_ref] + list(o_rest)
        pid = pl.program_id(0)
        for oref, v in zip(orefs, cols):
            @pl.when(pid == 0)
            def _(oref=oref):
                oref[...] = jnp.zeros((1, 1), jnp.float32)
            oref[...] += v.reshape(1, 1)

    scal = jax.ShapeDtypeStruct((1, 1), jnp.float32)
    sspec = pl.BlockSpec((1, 1), lambda i: (0, 0))
    return pl.pallas_call(
        body,
        out_shape=(scal,) * 6,
        grid=(G,),
        in_specs=[pl.BlockSpec((blk, EMB), lambda i: (i, 0)),
                  pl.BlockSpec((blk, EMB), lambda i: (i, 0)),
                  pl.BlockSpec((blk, EMB), lambda i: (i, 0)),
                  pl.BlockSpec((blk, NEG * EMB), lambda i: (i, 0)),
                  pl.BlockSpec((blk, NEG * EMB), lambda i: (i, 0))],
        out_specs=(sspec,) * 6,
    )(ue, pe, pc, ne2, nc2)


def _make_batch_gather():
    nch_u = BATCH // NS // CH
    nch_n = BATCH * NEG // NS // CH

    scratch = [
        pltpu.VMEM((nch_u, CH), jnp.int32),
        pltpu.VMEM((nch_u, CH), jnp.int32),
        pltpu.VMEM((nch_n, CH), jnp.int32),
        pltpu.VMEM((CH, 32), jnp.float32),
    ]
    out_t = (
        jax.ShapeDtypeStruct((BATCH, 2, 32), jnp.float32),
        jax.ShapeDtypeStruct((BATCH, 2, 32), jnp.float32),
        jax.ShapeDtypeStruct((BATCH, 2, 32), jnp.float32),
        jax.ShapeDtypeStruct((BATCH * NEG, 2, 32), jnp.float32),
        jax.ShapeDtypeStruct((BATCH * NEG, 2, 32), jnp.float32),
    )

    def body(ures, eres, ires, uidx, pidx, nidx,
             oue, ope, opc, one, onc, ub, pb, nb, rbuf):
        c = lax.axis_index("c")
        s = lax.axis_index("s")

        def load_transform(src_hbm, buf, nch):
            pltpu.sync_copy(src_hbm.at[pl.ds(s * nch, nch), :], buf)
            for r in range(nch):
                for k in range(CH // L):
                    v = buf[r, pl.ds(k * L, L)]
                    buf[r, pl.ds(k * L, L)] = v * 2 + c

        load_transform(uidx, ub, nch_u)
        load_transform(pidx, pb, nch_u)
        load_transform(nidx, nb, nch_n)

        def job(tbl, buf, nch, out):
            for j in range(nch):
                pltpu.sync_copy(tbl.at[buf.at[j]], rbuf)
                row = (s * nch + j) * CH

                @pl.when(c == 0)
                def _():
                    pltpu.sync_copy(rbuf, out.at[pl.ds(row, CH), 0, :])

                @pl.when(c == 1)
                def _():
                    pltpu.sync_copy(rbuf, out.at[pl.ds(row, CH), 1, :])

        job(ures, ub, nch_u, oue)
        job(eres, pb, nch_u, ope)
        job(ires, pb, nch_u, opc)
        job(eres, nb, nch_n, one)
        job(ires, nb, nch_n, onc)

    return pl.kernel(
        body,
        out_type=out_t,
        mesh=_mesh(),
        scratch_types=scratch,
        compiler_params=pltpu.CompilerParams(use_tc_tiling_on_sc=False, needs_layout_passes=False),
    )


def _make_angle(tpad):
    nch = tpad // (NC * NS) // CH

    scratch = [
        pltpu.VMEM((nch, CH), jnp.int32),
        pltpu.VMEM((nch, CH), jnp.int32),
        pltpu.VMEM((CH, EMB), jnp.float32),
        pltpu.VMEM((CH, EMB), jnp.float32),
        pltpu.VMEM((CH, 4), jnp.float32),
    ]

    def body(emb, hidx, tidx, out, hb, tb, hrows, trows, dots):
        c = lax.axis_index("c")
        s = lax.axis_index("s")
        w = s * NC + c

        pltpu.sync_copy(hidx.at[pl.ds(w * nch, nch), :], hb)
        pltpu.sync_copy(tidx.at[pl.ds(w * nch, nch), :], tb)

        def txf(r, _):
            for k in range(CH // L):
                v = hb[r, pl.ds(k * L, L)]
                hb[r, pl.ds(k * L, L)] = v + NU
                v2 = tb[r, pl.ds(k * L, L)]
                tb[r, pl.ds(k * L, L)] = v2 + NU
            return 0
        lax.fori_loop(0, nch, txf, 0)

        lanes = lax.iota(jnp.int32, L)

        def do_chunk(j, _):
            pltpu.sync_copy(emb.at[hb.at[j]], hrows)
            pltpu.sync_copy(emb.at[tb.at[j]], trows)
            for g in range(CH // L):
                rowi = lanes + g * L
                zero = jnp.zeros((L,), jnp.float32)

                def dim_step(d, carry):
                    hh, tt, ht = carry
                    ci = jnp.full((L,), 0, jnp.int32) + d
                    h = plsc.load_gather(hrows, [rowi, ci])
                    t = plsc.load_gather(trows, [rowi, ci])
                    return (hh + h * h, tt + t * t, ht + h * t)

                hh, tt, ht = lax.fori_loop(0, EMB, dim_step, (zero, zero, zero))
                plsc.store_scatter(dots, [rowi, jnp.full((L,), 0, jnp.int32)], hh)
                plsc.store_scatter(dots, [rowi, jnp.full((L,), 1, jnp.int32)], tt)
                plsc.store_scatter(dots, [rowi, jnp.full((L,), 2, jnp.int32)], ht)
            pltpu.sync_copy(dots, out.at[pl.ds((w * nch + j) * CH, CH), :])
            return 0

        lax.fori_loop(0, nch, do_chunk, 0)

    return pl.kernel(
        body,
        out_type=jax.ShapeDtypeStruct((tpad, 4), jnp.float32),
        mesh=_mesh(),
        scratch_types=scratch,
        compiler_params=pltpu.CompilerParams(use_tc_tiling_on_sc=False, needs_layout_passes=False),
    )


def _pad_edges(x, fill):
    pad = EPAD - x.shape[0]
    if x.dtype == jnp.float32:
        tailv = jnp.full((pad,), fill, jnp.float32)
        return jnp.concatenate([x, tailv]).reshape(-1, CH)
    tailv = jnp.full((pad,), fill, jnp.int32)
    return jnp.concatenate([x.astype(jnp.int32), tailv]).reshape(-1, CH)


def kernel(user, pos_item, neg_item, all_embed, item_emb_cf, rel_emb,
           edge_index, edge_type, ui_rows, ui_cols, ui_vals, tri_head, tri_tail):
    f32 = jnp.float32
    head = edge_index[0].astype(jnp.int32)
    tail = edge_index[1].astype(jnp.int32)
    et = edge_type.astype(jnp.int32)

    tail_p = _pad_edges(tail, 0)
    head_p = _pad_edges(head, NENT)
    et_p = _pad_edges(et, 1)
    cols_src = _pad_edges(ui_cols.astype(jnp.int32), 0)
    cols_dst = _pad_edges(ui_cols.astype(jnp.int32), NI)
    rows_src = _pad_edges(ui_rows.astype(jnp.int32), 0)
    rows_dst = _pad_edges(ui_rows.astype(jnp.int32), NU)
    vals_i = lax.bitcast_convert_type(_pad_edges(ui_vals.astype(f32), 0.0),
                                      jnp.int32)

    kg_edges = jnp.stack([tail_p, head_p, et_p], axis=1)
    ua_edges = jnp.stack([cols_src, rows_dst, vals_i], axis=1)
    icf_edges = jnp.stack([rows_src, cols_dst, vals_i], axis=1)
    ones_i = lax.bitcast_convert_type(_pad_edges(jnp.ones((NEDGE,), f32), 0.0),
                                      jnp.int32)
    deg_edges = jnp.stack([jnp.zeros_like(tail_p), head_p, ones_i], axis=1)

    rel2 = rel_emb.astype(f32).reshape(4 * (NREL - 1), 16)
    all2 = all_embed.astype(f32).reshape(4 * (NU + NENT), 16)
    icf2 = item_emb_cf.astype(f32).reshape(4 * NI, 16)

    ones_tbl = jnp.ones((4, 16), f32)
    seg_deg = _make_segsum(4, ACC_E, RPT_E, "val", 0)
    deg_full = seg_deg(ones_tbl, deg_edges)
    deg8 = deg_full[:, 0, 0:8]

    e_res = jnp.concatenate([all_embed[NU:], jnp.zeros((ACC_E - NENT, EMB), f32)])
    u_res = jnp.concatenate([all_embed[:NU], jnp.zeros((ACC_E - NU, EMB), f32)])
    i_res = jnp.concatenate([item_emb_cf, jnp.zeros((ACC_I - NI, EMB), f32)])

    seg_kg_0 = _make_segsum(4 * (NU + NENT), ACC_E, RPT_E, "rel", NU)
    seg_ua_0 = _make_segsum(4 * (NU + NENT), ACC_E, RPT_E, "val", NU)
    seg_kg = _make_segsum(4 * ACC_E, ACC_E, RPT_E, "rel", 0)
    seg_ua = _make_segsum(4 * ACC_E, ACC_E, RPT_E, "val", 0)
    seg_ucf_0 = _make_segsum(4 * NI, ACC_E, RPT_E, "val", 0)
    seg_ucf = _make_segsum(4 * ACC_I, ACC_E, RPT_E, "val", 0)
    seg_icf = _make_segsum(4 * ACC_E, ACC_I, RPT_I, "val", 0)

    ent2 = all2
    icf_t = icf2
    for hop in range(HOPS):
        if hop == 0:
            agg = seg_kg_0(ent2, kg_edges, rel2)
            uagg = seg_ua_0(ent2, ua_edges)
            ucf = seg_ucf_0(icf_t, ua_edges)
        else:
            agg = seg_kg(ent2, kg_edges, rel2)
            uagg = seg_ua(ent2, ua_edges)
            ucf = seg_ucf(icf_t, ua_edges)
        icf_new = seg_icf(ucf.reshape(4 * ACC_E, 16), icf_edges)

        ent_next, e_res = _div_norm_res(agg.reshape(ACC_E, EMB), deg8, e_res)
        u_res = _norm_res(uagg.reshape(ACC_E, EMB), u_res)
        i_res = _norm_res(icf_new.reshape(ACC_I, EMB), i_res)

        ent2 = ent_next.reshape(4 * ACC_E, 16)
        icf_t = icf_new.reshape(4 * ACC_I, 16)

    bg = _make_batch_gather()
    neg_flat = neg_item.reshape(-1).astype(jnp.int32)
    oue, ope, opc, one, onc = bg(
        u_res.reshape(2 * ACC_E, 32),
        e_res.reshape(2 * ACC_E, 32),
        i_res.reshape(2 * ACC_I, 32),
        user.astype(jnp.int32).reshape(-1, CH),
        pos_item.astype(jnp.int32).reshape(-1, CH),
        neg_flat.reshape(-1, CH))

    tot, su, sp, sne, spc, snc = _loss_partials(
        oue.reshape(BATCH, EMB),
        ope.reshape(BATCH, EMB),
        opc.reshape(BATCH, EMB),
        one.reshape(BATCH, NEG * EMB),
        onc.reshape(BATCH, NEG * EMB))
    loss1 = tot[0, 0] / BATCH
    reg = DECAY * (su + sp + sne + spc + snc)[0, 0] / (2.0 * BATCH)

    et_n = tri_head.shape[0]
    gran = NC * NS * CH
    tpad = max((et_n + gran - 1) // gran, 1) * gran
    th_p = jnp.concatenate(
        [tri_head.astype(jnp.int32),
         jnp.zeros((tpad - et_n,), jnp.int32)]).reshape(-1, CH)
    tt_p = jnp.concatenate(
        [tri_tail.astype(jnp.int32),
         jnp.zeros((tpad - et_n,), jnp.int32)]).reshape(-1, CH)
    ang = _make_angle(tpad)
    dots = ang(all_embed.astype(f32), th_p, tt_p)

    sc = ANGLE_DROP * ANGLE_DROP
    hh = dots[:et_n, 0] * sc
    tt = dots[:et_n, 1] * sc
    ht = dots[:et_n, 2] * sc
    eps = 1e-6
    nu_ = jnp.sqrt(hh)
    edist = jnp.sqrt(jnp.maximum(hh + tt - 2.0 * ht, 0.0))
    num = ht * (1.0 + hh) - hh * (1.0 + tt)
    denom = nu_ * edist * jnp.sqrt(jnp.clip(1.0 + tt * hh - 2.0 * ht, eps)) + eps
    angle = jnp.arccos(jnp.clip(num / denom, -1.0 + eps, 1.0 - eps))
    sqnu = jnp.clip(hh, 0.0, 1.0 - eps)
    half_ap = jnp.arcsin(jnp.clip(0.1 * (1.0 - sqnu) / jnp.sqrt(sqnu + eps),
                                  -1.0 + eps, 1.0 - eps))
    loss2 = ANGLE_W * jnp.sum(jnp.maximum(angle - half_ap, 0.0)) / et_n

    return loss1 + reg + loss2


# speedup vs baseline: 1.1089x; 1.1089x over previous
import jax
import jax.numpy as jnp
from jax import lax
from jax.experimental import pallas as pl
from jax.experimental.pallas import tpu as pltpu
from jax.experimental.pallas import tpu_sc as plsc

NU = 50000
NI = 20000
NENT = 50000
NREL = 17
EMB = 64
HOPS = 2
NEDGE = 800000
BATCH = 4096
NEG = 16
MARGIN = 0.8
DECAY = 1e-4
ANGLE_W = 0.5
ANGLE_DROP = 0.5

NC, NS, L = 2, 16, 16
CH = 128
SB = 56
NSB = 7
EPT = CH * SB * NSB
EPAD = EPT * NS

NR = EPAD // CH
ACC_E = 50176
RPT_E = ACC_E // NS
ACC_I = 20480
RPT_I = ACC_I // NS


def _mesh():
    return plsc.VectorSubcoreMesh(core_axis_name="c", subcore_axis_name="s")


def _make_segsum(tbl_rows4, accr, rpt, mode, off):
    scratch = [
        pltpu.VMEM((SB, 3, CH), jnp.int32),
        pltpu.VMEM((2, CH, L), jnp.float32),
        pltpu.VMEM((112, L), jnp.float32),
        pltpu.VMEM_SHARED((accr, L), jnp.float32),
        pltpu.SemaphoreType.DMA,
        pltpu.SemaphoreType.DMA,
    ]
    if mode == "rel":
        scratch.insert(1, pltpu.VMEM((2, CH, L), jnp.float32))
        scratch.append(pltpu.SemaphoreType.DMA)
        scratch.append(pltpu.SemaphoreType.DMA)

    def body(tbl, edges, *rest):
        if mode == "rel":
            rel, out, esb, relrows, rows, zbuf, acc, sem0, sem1, sr0, sr1 = rest
            rsems = [sr0, sr1]
        else:
            out, esb, rows, zbuf, acc, sem0, sem1 = rest
            relrows = None
        c = lax.axis_index("c")
        s = lax.axis_index("s")
        zv = jnp.zeros((L,), jnp.float32)
        for zi in range(112):
            zbuf[zi, pl.ds(0, L)] = zv

        row0 = s * (SB * NSB)
        sems = [sem0, sem1]

        for p in range(2):
            q = c + 2 * p

            def zloop(k, _):
                pltpu.sync_copy(zbuf, acc.at[pl.ds(s * rpt + k * 112, 112), :])
                return 0
            lax.fori_loop(0, rpt // 112, zloop, 0)
            plsc.subcore_barrier()

            def chunk(j, b):
                pltpu.make_async_copy(tbl.at[esb.at[j, 0]], rows.at[b],
                                      sems[b]).wait()
                if mode == "rel":
                    pltpu.make_async_copy(rel.at[esb.at[j, 2]], relrows.at[b],
                                          rsems[b]).wait()

                    def grp(g, _):
                        i0 = g * L
                        for kk in range(L):
                            r_ = rows[b, i0 + kk, pl.ds(0, L)]
                            m_ = relrows[b, i0 + kk, pl.ds(0, L)]
                            rows[b, i0 + kk, pl.ds(0, L)] = r_ * m_
                        return 0
                else:
                    def grp(g, _):
                        i0 = g * L
                        vv = plsc.bitcast(esb[j, 2, pl.ds(i0, L)], jnp.float32)
                        for kk in range(L):
                            v = vv[kk]
                            r_ = rows[b, i0 + kk, pl.ds(0, L)]
                            rows[b, i0 + kk, pl.ds(0, L)] = r_ * v
                        return 0
                lax.fori_loop(0, CH // L, grp, 0)
                pltpu.sync_copy(rows.at[b], acc.at[esb.at[j, 1]], add=True)

            def super_chunk(sb, _):
                r0 = row0 + sb * SB
                pltpu.sync_copy(edges.at[pl.ds(r0, SB), :, :], esb)

                def txf(r, _):
                    for k in range(CH // L):
                        v = esb[r, 0, pl.ds(k * L, L)]
                        esb[r, 0, pl.ds(k * L, L)] = v * 4 + (q + 4 * off)
                        if mode == "rel":
                            t_ = esb[r, 2, pl.ds(k * L, L)]
                            esb[r, 2, pl.ds(k * L, L)] = t_ * 4 + (q - 4)
                    return 0
                lax.fori_loop(0, SB, txf, 0)

                pltpu.async_copy(tbl.at[esb.at[0, 0]], rows.at[0], sem0)
                pltpu.async_copy(tbl.at[esb.at[1, 0]], rows.at[1], sem1)
                if mode == "rel":
                    pltpu.async_copy(rel.at[esb.at[0, 2]], relrows.at[0], sr0)
                    pltpu.async_copy(rel.at[esb.at[1, 2]], relrows.at[1], sr1)

                def loop_body(m, _):
                    j0 = 2 * m
                    chunk(j0, 0)

                    @pl.when(j0 + 2 < SB)
                    def _():
                        pltpu.async_copy(tbl.at[esb.at[j0 + 2, 0]], rows.at[0],
                                         sem0)
                        if mode == "rel":
                            pltpu.async_copy(rel.at[esb.at[j0 + 2, 2]],
                                             relrows.at[0], sr0)

                    chunk(j0 + 1, 1)

                    @pl.when(j0 + 3 < SB)
                    def _():
                        pltpu.async_copy(tbl.at[esb.at[j0 + 3, 0]], rows.at[1],
                                         sem1)
                        if mode == "rel":
                            pltpu.async_copy(rel.at[esb.at[j0 + 3, 2]],
                                             relrows.at[1], sr1)
                    return 0

                lax.fori_loop(0, SB // 2, loop_body, 0)
                return 0

            lax.fori_loop(0, NSB, super_chunk, 0)
            plsc.subcore_barrier()

            pltpu.sync_copy(acc.at[pl.ds(s * rpt, rpt), :],
                            out.at[pl.ds(s * rpt, rpt), q, :])
            plsc.subcore_barrier()

    del tbl_rows4
    return pl.kernel(
        body,
        out_type=jax.ShapeDtypeStruct((accr, 4, L), jnp.float32),
        mesh=_mesh(),
        scratch_types=scratch,
        compiler_params=pltpu.CompilerParams(use_tc_tiling_on_sc=False,
                                             needs_layout_passes=False),
    )


def _norm_res(x, res):
    R = x.shape[0]
    blk = 512

    def body(x_ref, r_ref, o_ref):
        xv = x_ref[...]
        nrm = jnp.sqrt(jnp.sum(xv * xv, axis=1, keepdims=True))
        o_ref[...] = r_ref[...] + xv / (nrm + 1e-8)

    return pl.pallas_call(
        body,
        out_shape=jax.ShapeDtypeStruct((R, EMB), jnp.float32),
        grid=(R // blk,),
        in_specs=[pl.BlockSpec((blk, EMB), lambda i: (i, 0)),
                  pl.BlockSpec((blk, EMB), lambda i: (i, 0))],
        out_specs=pl.BlockSpec((blk, EMB), lambda i: (i, 0)),
    )(x, res)


def _div_norm_res(x, deg8, res):
    R = x.shape[0]
    blk = 512

    def body(x_ref, d_ref, r_ref, e_ref, o_ref):
        xv = x_ref[...]
        d = jnp.maximum(d_ref[:, 0:1], 1.0)
        e_ref[...] = xv / d
        nrm = jnp.sqrt(jnp.sum(xv * xv, axis=1, keepdims=True))
        o_ref[...] = r_ref[...] + xv / (nrm + 1e-8)

    return pl.pallas_call(
        body,
        out_shape=(jax.ShapeDtypeStruct((R, EMB), jnp.float32),
                   jax.ShapeDtypeStruct((R, EMB), jnp.float32)),
        grid=(R // blk,),
        in_specs=[pl.BlockSpec((blk, EMB), lambda i: (i, 0)),
                  pl.BlockSpec((blk, 8), lambda i: (i, 0)),
                  pl.BlockSpec((blk, EMB), lambda i: (i, 0))],
        out_specs=(pl.BlockSpec((blk, EMB), lambda i: (i, 0)),
                   pl.BlockSpec((blk, EMB), lambda i: (i, 0))),
    )(x, deg8, res)


def _loss_partials(ue, pe, pc, ne2, nc2):
    blk = 512
    G = BATCH // blk

    def body(u_ref, p_ref, pc_ref, n_ref, nc_ref, o_ref, *o_rest):
        u_raw = u_ref[...]
        p_raw = p_ref[...]
        pc_raw = pc_ref[...]
        u = u_raw / (jnp.sqrt(jnp.sum(u_raw * u_raw, axis=1, keepdims=True)) + 1e-8)
        ps = p_raw + pc_raw
        p = ps / (jnp.sqrt(jnp.sum(ps * ps, axis=1, keepdims=True)) + 1e-8)
        pos_score = jnp.sum(u * p, axis=1, keepdims=True)
        hinge = jnp.maximum(1.0 - pos_score, 0.0)
        neg_acc = jnp.zeros((blk, 1), jnp.float32)
        sq_ne = jnp.zeros((), jnp.float32)
        sq_nc = jnp.zeros((), jnp.float32)
        for k in range(NEG):
            nk_raw = n_ref[:, k * EMB:(k + 1) * EMB]
            nck_raw = nc_ref[:, k * EMB:(k + 1) * EMB]
            sq_ne += jnp.sum(nk_raw * nk_raw)
            sq_nc += jnp.sum(nck_raw * nck_raw)
            nk = nk_raw + nck_raw
            nkn = nk / (jnp.sqrt(jnp.sum(nk * nk, axis=1, keepdims=True)) + 1e-8)
            ns = jnp.sum(u * nkn, axis=1, keepdims=True)
            neg_acc += jnp.maximum(ns - MARGIN, 0.0)
        tot = jnp.sum(hinge + neg_acc * (1.0 / NEG))
        cols = [tot,
                jnp.sum(u_raw * u_raw),
                jnp.sum(p_raw * p_raw),
                sq_ne,
                jnp.sum(pc_raw * pc_raw),
                sq_nc]
        orefs = [o_ref] + list(o_rest)
        pid = pl.program_id(0)
        for oref, v in zip(orefs, cols):
            @pl.when(pid == 0)
            def _(oref=oref):
                oref[...] = jnp.zeros((1, 1), jnp.float32)
            oref[...] += v.reshape(1, 1)

    scal = jax.ShapeDtypeStruct((1, 1), jnp.float32)
    sspec = pl.BlockSpec((1, 1), lambda i: (0, 0))
    return pl.pallas_call(
        body,
        out_shape=(scal,) * 6,
        grid=(G,),
        in_specs=[pl.BlockSpec((blk, EMB), lambda i: (i, 0)),
                  pl.BlockSpec((blk, EMB), lambda i: (i, 0)),
                  pl.BlockSpec((blk, EMB), lambda i: (i, 0)),
                  pl.BlockSpec((blk, NEG * EMB), lambda i: (i, 0)),
                  pl.BlockSpec((blk, NEG * EMB), lambda i: (i, 0))],
        out_specs=(sspec,) * 6,
    )(ue, pe, pc, ne2, nc2)


def _make_batch_gather():
    nch_u = BATCH // NS // CH
    nch_n = BATCH * NEG // NS // CH

    scratch = [
        pltpu.VMEM((nch_u, CH), jnp.int32),
        pltpu.VMEM((nch_u, CH), jnp.int32),
        pltpu.VMEM((nch_n, CH), jnp.int32),
        pltpu.VMEM((CH, 32), jnp.float32),
    ]
    out_t = (
        jax.ShapeDtypeStruct((BATCH, 2, 32), jnp.float32),
        jax.ShapeDtypeStruct((BATCH, 2, 32), jnp.float32),
        jax.ShapeDtypeStruct((BATCH, 2, 32), jnp.float32),
        jax.ShapeDtypeStruct((BATCH * NEG, 2, 32), jnp.float32),
        jax.ShapeDtypeStruct((BATCH * NEG, 2, 32), jnp.float32),
    )

    def body(ures, eres, ires, uidx, pidx, nidx,
             oue, ope, opc, one, onc, ub, pb, nb, rbuf):
        c = lax.axis_index("c")
        s = lax.axis_index("s")

        def load_transform(src_hbm, buf, nch):
            pltpu.sync_copy(src_hbm.at[pl.ds(s * nch, nch), :], buf)
            for r in range(nch):
                for k in range(CH // L):
                    v = buf[r, pl.ds(k * L, L)]
                    buf[r, pl.ds(k * L, L)] = v * 2 + c

        load_transform(uidx, ub, nch_u)
        load_transform(pidx, pb, nch_u)
        load_transform(nidx, nb, nch_n)

        def job(tbl, buf, nch, out):
            for j in range(nch):
                pltpu.sync_copy(tbl.at[buf.at[j]], rbuf)
                row = (s * nch + j) * CH

                @pl.when(c == 0)
                def _():
                    pltpu.sync_copy(rbuf, out.at[pl.ds(row, CH), 0, :])

                @pl.when(c == 1)
                def _():
                    pltpu.sync_copy(rbuf, out.at[pl.ds(row, CH), 1, :])

        job(ures, ub, nch_u, oue)
        job(eres, pb, nch_u, ope)
        job(ires, pb, nch_u, opc)
        job(eres, nb, nch_n, one)
        job(ires, nb, nch_n, onc)

    return pl.kernel(
        body,
        out_type=out_t,
        mesh=_mesh(),
        scratch_types=scratch,
        compiler_params=pltpu.CompilerParams(use_tc_tiling_on_sc=False, needs_layout_passes=False),
    )


def _make_angle(tpad):
    nch = tpad // (NC * NS) // CH

    scratch = [
        pltpu.VMEM((nch, CH), jnp.int32),
        pltpu.VMEM((nch, CH), jnp.int32),
        pltpu.VMEM((CH, EMB), jnp.float32),
        pltpu.VMEM((CH, EMB), jnp.float32),
        pltpu.VMEM((CH, 4), jnp.float32),
    ]

    def body(emb, hidx, tidx, out, hb, tb, hrows, trows, dots):
        c = lax.axis_index("c")
        s = lax.axis_index("s")
        w = s * NC + c

        pltpu.sync_copy(hidx.at[pl.ds(w * nch, nch), :], hb)
        pltpu.sync_copy(tidx.at[pl.ds(w * nch, nch), :], tb)

        def txf(r, _):
            for k in range(CH // L):
                v = hb[r, pl.ds(k * L, L)]
                hb[r, pl.ds(k * L, L)] = v + NU
                v2 = tb[r, pl.ds(k * L, L)]
                tb[r, pl.ds(k * L, L)] = v2 + NU
            return 0
        lax.fori_loop(0, nch, txf, 0)

        lanes = lax.iota(jnp.int32, L)

        def do_chunk(j, _):
            pltpu.sync_copy(emb.at[hb.at[j]], hrows)
            pltpu.sync_copy(emb.at[tb.at[j]], trows)
            for g in range(CH // L):
                rowi = lanes + g * L
                zero = jnp.zeros((L,), jnp.float32)

                def dim_step(d, carry):
                    hh, tt, ht = carry
                    ci = jnp.full((L,), 0, jnp.int32) + d
                    h = plsc.load_gather(hrows, [rowi, ci])
                    t = plsc.load_gather(trows, [rowi, ci])
                    return (hh + h * h, tt + t * t, ht + h * t)

                hh, tt, ht = lax.fori_loop(0, EMB, dim_step, (zero, zero, zero))
                plsc.store_scatter(dots, [rowi, jnp.full((L,), 0, jnp.int32)], hh)
                plsc.store_scatter(dots, [rowi, jnp.full((L,), 1, jnp.int32)], tt)
                plsc.store_scatter(dots, [rowi, jnp.full((L,), 2, jnp.int32)], ht)
            pltpu.sync_copy(dots, out.at[pl.ds((w * nch + j) * CH, CH), :])
            return 0

        lax.fori_loop(0, nch, do_chunk, 0)

    return pl.kernel(
        body,
        out_type=jax.ShapeDtypeStruct((tpad, 4), jnp.float32),
        mesh=_mesh(),
        scratch_types=scratch,
        compiler_params=pltpu.CompilerParams(use_tc_tiling_on_sc=False, needs_layout_passes=False),
    )


def _pad_edges(x, fill):
    pad = EPAD - x.shape[0]
    if x.dtype == jnp.float32:
        tailv = jnp.full((pad,), fill, jnp.float32)
        return jnp.concatenate([x, tailv]).reshape(-1, CH)
    tailv = jnp.full((pad,), fill, jnp.int32)
    return jnp.concatenate([x.astype(jnp.int32), tailv]).reshape(-1, CH)


def kernel(user, pos_item, neg_item, all_embed, item_emb_cf, rel_emb,
           edge_index, edge_type, ui_rows, ui_cols, ui_vals, tri_head, tri_tail):
    f32 = jnp.float32
    head = edge_index[0].astype(jnp.int32)
    tail = edge_index[1].astype(jnp.int32)
    et = edge_type.astype(jnp.int32)

    tail_p = _pad_edges(tail, 0)
    head_p = _pad_edges(head, NENT)
    et_p = _pad_edges(et, 1)
    cols_src = _pad_edges(ui_cols.astype(jnp.int32), 0)
    cols_dst = _pad_edges(ui_cols.astype(jnp.int32), NI)
    rows_src = _pad_edges(ui_rows.astype(jnp.int32), 0)
    rows_dst = _pad_edges(ui_rows.astype(jnp.int32), NU)
    vals_i = lax.bitcast_convert_type(_pad_edges(ui_vals.astype(f32), 0.0),
                                      jnp.int32)

    kg_edges = jnp.stack([tail_p, head_p, et_p], axis=1)
    ua_edges = jnp.stack([cols_src, rows_dst, vals_i], axis=1)
    icf_edges = jnp.stack([rows_src, cols_dst, vals_i], axis=1)
    ones_i = lax.bitcast_convert_type(_pad_edges(jnp.ones((NEDGE,), f32), 0.0),
                                      jnp.int32)
    deg_edges = jnp.stack([jnp.zeros_like(tail_p), head_p, ones_i], axis=1)

    rel2 = rel_emb.astype(f32).reshape(4 * (NREL - 1), 16)
    all2 = all_embed.astype(f32).reshape(4 * (NU + NENT), 16)
    icf2 = item_emb_cf.astype(f32).reshape(4 * NI, 16)

    ones_tbl = jnp.ones((4, 16), f32)
    seg_deg = _make_segsum(4, ACC_E, RPT_E, "val", 0)
    deg_full = seg_deg(ones_tbl, deg_edges)
    deg8 = deg_full[:, 0, 0:8]

    e_res = jnp.concatenate([all_embed[NU:], jnp.zeros((ACC_E - NENT, EMB), f32)])
    u_res = jnp.concatenate([all_embed[:NU], jnp.zeros((ACC_E - NU, EMB), f32)])
    i_res = jnp.concatenate([item_emb_cf, jnp.zeros((ACC_I - NI, EMB), f32)])

    seg_kg_0 = _make_segsum(4 * (NU + NENT), ACC_E, RPT_E, "rel", NU)
    seg_ua_0 = _make_segsum(4 * (NU + NENT), ACC_E, RPT_E, "val", NU)
    seg_kg = _make_segsum(4 * ACC_E, ACC_E, RPT_E, "rel", 0)
    seg_ua = _make_segsum(4 * ACC_E, ACC_E, RPT_E, "val", 0)
    seg_ucf_0 = _make_segsum(4 * NI, ACC_E, RPT_E, "val", 0)
    seg_ucf = _make_segsum(4 * ACC_I, ACC_E, RPT_E, "val", 0)
    seg_icf = _make_segsum(4 * ACC_E, ACC_I, RPT_I, "val", 0)

    ent2 = all2
    icf_t = icf2
    for hop in range(HOPS):
        if hop == 0:
            agg = seg_kg_0(ent2, kg_edges, rel2)
            uagg = seg_ua_0(ent2, ua_edges)
            ucf = seg_ucf_0(icf_t, ua_edges)
        else:
            agg = seg_kg(ent2, kg_edges, rel2)
            uagg = seg_ua(ent2, ua_edges)
            ucf = seg_ucf(icf_t, ua_edges)
        icf_new = seg_icf(ucf.reshape(4 * ACC_E, 16), icf_edges)

        ent_next, e_res = _div_norm_res(agg.reshape(ACC_E, EMB), deg8, e_res)
        u_res = _norm_res(uagg.reshape(ACC_E, EMB), u_res)
        i_res = _norm_res(icf_new.reshape(ACC_I, EMB), i_res)

        ent2 = ent_next.reshape(4 * ACC_E, 16)
        icf_t = icf_new.reshape(4 * ACC_I, 16)

    bg = _make_batch_gather()
    neg_flat = neg_item.reshape(-1).astype(jnp.int32)
    oue, ope, opc, one, onc = bg(
        u_res.reshape(2 * ACC_E, 32),
        e_res.reshape(2 * ACC_E, 32),
        i_res.reshape(2 * ACC_I, 32),
        user.astype(jnp.int32).reshape(-1, CH),
        pos_item.astype(jnp.int32).reshape(-1, CH),
        neg_flat.reshape(-1, CH))

    tot, su, sp, sne, spc, snc = _loss_partials(
        oue.reshape(BATCH, EMB),
        ope.reshape(BATCH, EMB),
        opc.reshape(BATCH, EMB),
        one.reshape(BATCH, NEG * EMB),
        onc.reshape(BATCH, NEG * EMB))
    loss1 = tot[0, 0] / BATCH
    reg = DECAY * (su + sp + sne + spc + snc)[0, 0] / (2.0 * BATCH)

    et_n = tri_head.shape[0]
    gran = NC * NS * CH
    tpad = max((et_n + gran - 1) // gran, 1) * gran
    th_p = jnp.concatenate(
        [tri_head.astype(jnp.int32),
         jnp.zeros((tpad - et_n,), jnp.int32)]).reshape(-1, CH)
    tt_p = jnp.concatenate(
        [tri_tail.astype(jnp.int32),
         jnp.zeros((tpad - et_n,), jnp.int32)]).reshape(-1, CH)
    ang = _make_angle(tpad)
    dots = ang(all_embed.astype(f32), th_p, tt_p)

    sc = ANGLE_DROP * ANGLE_DROP
    hh = dots[:et_n, 0] * sc
    tt = dots[:et_n, 1] * sc
    ht = dots[:et_n, 2] * sc
    eps = 1e-6
    nu_ = jnp.sqrt(hh)
    edist = jnp.sqrt(jnp.maximum(hh + tt - 2.0 * ht, 0.0))
    num = ht * (1.0 + hh) - hh * (1.0 + tt)
    denom = nu_ * edist * jnp.sqrt(jnp.clip(1.0 + tt * hh - 2.0 * ht, eps)) + eps
    angle = jnp.arccos(jnp.clip(num / denom, -1.0 + eps, 1.0 - eps))
    sqnu = jnp.clip(hh, 0.0, 1.0 - eps)
    half_ap = jnp.arcsin(jnp.clip(0.1 * (1.0 - sqnu) / jnp.sqrt(sqnu + eps),
                                  -1.0 + eps, 1.0 - eps))
    loss2 = ANGLE_W * jnp.sum(jnp.maximum(angle - half_ap, 0.0)) / et_n

    return loss1 + reg + loss2


# speedup vs baseline: 4.1394x; 3.7329x over previous
import jax
import jax.numpy as jnp
from jax import lax
from jax.experimental import pallas as pl
from jax.experimental.pallas import tpu as pltpu
from jax.experimental.pallas import tpu_sc as plsc

NU = 50000
NI = 20000
NENT = 50000
NREL = 17
EMB = 64
HOPS = 2
NEDGE = 800000
BATCH = 4096
NEG = 16
MARGIN = 0.8
DECAY = 1e-4
ANGLE_W = 0.5
ANGLE_DROP = 0.5

NC, NS, L = 2, 16, 16
CH = 128
SB = 56
NSB = 7
EPT = CH * SB * NSB
EPAD = EPT * NS

NR = EPAD // CH
ACC_E = 50176
RPT_E = ACC_E // NS
ACC_I = 20480
RPT_I = ACC_I // NS


def _mesh():
    return plsc.VectorSubcoreMesh(core_axis_name="c", subcore_axis_name="s")


def _make_segsum(tbl_rows4, accr, rpt, mode, off):
    scratch = [
        pltpu.VMEM((SB, 3, CH), jnp.int32),
        pltpu.VMEM((2, CH, L), jnp.float32),
        pltpu.VMEM((112, L), jnp.float32),
        pltpu.VMEM_SHARED((accr, L), jnp.float32),
        pltpu.SemaphoreType.DMA,
        pltpu.SemaphoreType.DMA,
    ]
    if mode == "rel":
        scratch.insert(1, pltpu.VMEM((4 * (NREL - 1), L), jnp.float32))

    def body(tbl, edges, *rest):
        if mode == "rel":
            rel, out, esb, relv, rows, zbuf, acc, sem0, sem1 = rest
        else:
            out, esb, rows, zbuf, acc, sem0, sem1 = rest
            relv = None
        c = lax.axis_index("c")
        s = lax.axis_index("s")
        zv = jnp.zeros((L,), jnp.float32)
        for zi in range(112):
            zbuf[zi, pl.ds(0, L)] = zv
        if mode == "rel":
            pltpu.sync_copy(rel, relv)

        row0 = s * (SB * NSB)
        sems = [sem0, sem1]
        if mode == "deg":
            ov = jnp.full((L,), 1.0, jnp.float32)
            for zi in range(CH):
                rows[0, zi, pl.ds(0, L)] = ov

        for p in range(1 if mode == "deg" else 2):
            q = c + 2 * p

            def zloop(k, _):
                pltpu.sync_copy(zbuf, acc.at[pl.ds(s * rpt + k * 112, 112), :])
                return 0
            lax.fori_loop(0, rpt // 112, zloop, 0)
            plsc.subcore_barrier()

            def chunk(j, b):
                if mode != "deg":
                    pltpu.make_async_copy(tbl.at[esb.at[j, 0]], rows.at[b],
                                          sems[b]).wait()
                if mode == "rel":
                    def grp(g, _):
                        i0 = g * L
                        tv = esb[j, 2, pl.ds(i0, L)]
                        for kk in range(L):
                            rr = 4 * tv[kk] + (q - 4)
                            relq = relv[rr, pl.ds(0, L)]
                            r_ = rows[b, i0 + kk, pl.ds(0, L)]
                            rows[b, i0 + kk, pl.ds(0, L)] = r_ * relq
                        return 0
                    lax.fori_loop(0, CH // L, grp, 0)
                elif mode == "val":
                    def grp(g, _):
                        i0 = g * L
                        vv = plsc.bitcast(esb[j, 2, pl.ds(i0, L)], jnp.float32)
                        for kk in range(L):
                            v = vv[kk]
                            r_ = rows[b, i0 + kk, pl.ds(0, L)]
                            rows[b, i0 + kk, pl.ds(0, L)] = r_ * v
                        return 0
                    lax.fori_loop(0, CH // L, grp, 0)
                pltpu.sync_copy(rows.at[b if mode != "deg" else 0],
                                acc.at[esb.at[j, 1]], add=True)

            def super_chunk(sb, _):
                r0 = row0 + sb * SB
                pltpu.sync_copy(edges.at[pl.ds(r0, SB), :, :], esb)

                if mode != "deg":
                    def txf(r, _):
                        for k in range(CH // L):
                            v = esb[r, 0, pl.ds(k * L, L)]
                            esb[r, 0, pl.ds(k * L, L)] = v * 4 + (q + 4 * off)
                        return 0
                    lax.fori_loop(0, SB, txf, 0)

                    pltpu.async_copy(tbl.at[esb.at[0, 0]], rows.at[0], sem0)
                    pltpu.async_copy(tbl.at[esb.at[1, 0]], rows.at[1], sem1)

                def loop_body(m, _):
                    j0 = 2 * m
                    chunk(j0, 0)

                    if mode != "deg":
                        @pl.when(j0 + 2 < SB)
                        def _():
                            pltpu.async_copy(tbl.at[esb.at[j0 + 2, 0]],
                                             rows.at[0], sem0)

                    chunk(j0 + 1, 1)

                    if mode != "deg":
                        @pl.when(j0 + 3 < SB)
                        def _():
                            pltpu.async_copy(tbl.at[esb.at[j0 + 3, 0]],
                                             rows.at[1], sem1)
                    return 0

                lax.fori_loop(0, SB // 2, loop_body, 0)
                return 0

            lax.fori_loop(0, NSB, super_chunk, 0)
            plsc.subcore_barrier()

            pltpu.sync_copy(acc.at[pl.ds(s * rpt, rpt), :],
                            out.at[pl.ds(s * rpt, rpt), q, :])
            plsc.subcore_barrier()

    del tbl_rows4
    return pl.kernel(
        body,
        out_type=jax.ShapeDtypeStruct((accr, 4, L), jnp.float32),
        mesh=_mesh(),
        scratch_types=scratch,
        compiler_params=pltpu.CompilerParams(use_tc_tiling_on_sc=False,
                                             needs_layout_passes=False),
    )


def _norm_res(x, res):
    R = x.shape[0]
    blk = 512

    def body(x_ref, r_ref, o_ref):
        xv = x_ref[...]
        nrm = jnp.sqrt(jnp.sum(xv * xv, axis=1, keepdims=True))
        o_ref[...] = r_ref[...] + xv / (nrm + 1e-8)

    return pl.pallas_call(
        body,
        out_shape=jax.ShapeDtypeStruct((R, EMB), jnp.float32),
        grid=(R // blk,),
        in_specs=[pl.BlockSpec((blk, EMB), lambda i: (i, 0)),
                  pl.BlockSpec((blk, EMB), lambda i: (i, 0))],
        out_specs=pl.BlockSpec((blk, EMB), lambda i: (i, 0)),
    )(x, res)


def _div_norm_res(x, deg8, res):
    R = x.shape[0]
    blk = 512

    def body(x_ref, d_ref, r_ref, e_ref, o_ref):
        xv = x_ref[...]
        d = jnp.maximum(d_ref[:, 0:1], 1.0)
        e_ref[...] = xv / d
        nrm = jnp.sqrt(jnp.sum(xv * xv, axis=1, keepdims=True))
        o_ref[...] = r_ref[...] + xv / (nrm + 1e-8)

    return pl.pallas_call(
        body,
        out_shape=(jax.ShapeDtypeStruct((R, EMB), jnp.float32),
                   jax.ShapeDtypeStruct((R, EMB), jnp.float32)),
        grid=(R // blk,),
        in_specs=[pl.BlockSpec((blk, EMB), lambda i: (i, 0)),
                  pl.BlockSpec((blk, 8), lambda i: (i, 0)),
                  pl.BlockSpec((blk, EMB), lambda i: (i, 0))],
        out_specs=(pl.BlockSpec((blk, EMB), lambda i: (i, 0)),
                   pl.BlockSpec((blk, EMB), lambda i: (i, 0))),
    )(x, deg8, res)


def _loss_partials(ue, pe, pc, ne2, nc2):
    blk = 512
    G = BATCH // blk

    def body(u_ref, p_ref, pc_ref, n_ref, nc_ref, o_ref, *o_rest):
        u_raw = u_ref[...]
        p_raw = p_ref[...]
        pc_raw = pc_ref[...]
        u = u_raw / (jnp.sqrt(jnp.sum(u_raw * u_raw, axis=1, keepdims=True)) + 1e-8)
        ps = p_raw + pc_raw
        p = ps / (jnp.sqrt(jnp.sum(ps * ps, axis=1, keepdims=True)) + 1e-8)
        pos_score = jnp.sum(u * p, axis=1, keepdims=True)
        hinge = jnp.maximum(1.0 - pos_score, 0.0)
        neg_acc = jnp.zeros((blk, 1), jnp.float32)
        sq_ne = jnp.zeros((), jnp.float32)
        sq_nc = jnp.zeros((), jnp.float32)
        for k in range(NEG):
            nk_raw = n_ref[:, k * EMB:(k + 1) * EMB]
            nck_raw = nc_ref[:, k * EMB:(k + 1) * EMB]
            sq_ne += jnp.sum(nk_raw * nk_raw)
            sq_nc += jnp.sum(nck_raw * nck_raw)
            nk = nk_raw + nck_raw
            nkn = nk / (jnp.sqrt(jnp.sum(nk * nk, axis=1, keepdims=True)) + 1e-8)
            ns = jnp.sum(u * nkn, axis=1, keepdims=True)
            neg_acc += jnp.maximum(ns - MARGIN, 0.0)
        tot = jnp.sum(hinge + neg_acc * (1.0 / NEG))
        cols = [tot,
                jnp.sum(u_raw * u_raw),
                jnp.sum(p_raw * p_raw),
                sq_ne,
                jnp.sum(pc_raw * pc_raw),
                sq_nc]
        orefs = [o_ref] + list(o_rest)
        pid = pl.program_id(0)
        for oref, v in zip(orefs, cols):
            @pl.when(pid == 0)
            def _(oref=oref):
                oref[...] = jnp.zeros((1, 1), jnp.float32)
            oref[...] += v.reshape(1, 1)

    scal = jax.ShapeDtypeStruct((1, 1), jnp.float32)
    sspec = pl.BlockSpec((1, 1), lambda i: (0, 0))
    return pl.pallas_call(
        body,
        out_shape=(scal,) * 6,
        grid=(G,),
        in_specs=[pl.BlockSpec((blk, EMB), lambda i: (i, 0)),
                  pl.BlockSpec((blk, EMB), lambda i: (i, 0)),
                  pl.BlockSpec((blk, EMB), lambda i: (i, 0)),
                  pl.BlockSpec((blk, NEG * EMB), lambda i: (i, 0)),
                  pl.BlockSpec((blk, NEG * EMB), lambda i: (i, 0))],
        out_specs=(sspec,) * 6,
    )(ue, pe, pc, ne2, nc2)


def _make_batch_gather():
    nch_u = BATCH // NS // CH
    nch_n = BATCH * NEG // NS // CH

    scratch = [
        pltpu.VMEM((nch_u, CH), jnp.int32),
        pltpu.VMEM((nch_u, CH), jnp.int32),
        pltpu.VMEM((nch_n, CH), jnp.int32),
        pltpu.VMEM((CH, 32), jnp.float32),
    ]
    out_t = (
        jax.ShapeDtypeStruct((BATCH, 2, 32), jnp.float32),
        jax.ShapeDtypeStruct((BATCH, 2, 32), jnp.float32),
        jax.ShapeDtypeStruct((BATCH, 2, 32), jnp.float32),
        jax.ShapeDtypeStruct((BATCH * NEG, 2, 32), jnp.float32),
        jax.ShapeDtypeStruct((BATCH * NEG, 2, 32), jnp.float32),
    )

    def body(ures, eres, ires, uidx, pidx, nidx,
             oue, ope, opc, one, onc, ub, pb, nb, rbuf):
        c = lax.axis_index("c")
        s = lax.axis_index("s")

        def load_transform(src_hbm, buf, nch):
            pltpu.sync_copy(src_hbm.at[pl.ds(s * nch, nch), :], buf)
            for r in range(nch):
                for k in range(CH // L):
                    v = buf[r, pl.ds(k * L, L)]
                    buf[r, pl.ds(k * L, L)] = v * 2 + c

        load_transform(uidx, ub, nch_u)
        load_transform(pidx, pb, nch_u)
        load_transform(nidx, nb, nch_n)

        def job(tbl, buf, nch, out):
            for j in range(nch):
                pltpu.sync_copy(tbl.at[buf.at[j]], rbuf)
                row = (s * nch + j) * CH

                @pl.when(c == 0)
                def _():
                    pltpu.sync_copy(rbuf, out.at[pl.ds(row, CH), 0, :])

                @pl.when(c == 1)
                def _():
                    pltpu.sync_copy(rbuf, out.at[pl.ds(row, CH), 1, :])

        job(ures, ub, nch_u, oue)
        job(eres, pb, nch_u, ope)
        job(ires, pb, nch_u, opc)
        job(eres, nb, nch_n, one)
        job(ires, nb, nch_n, onc)

    return pl.kernel(
        body,
        out_type=out_t,
        mesh=_mesh(),
        scratch_types=scratch,
        compiler_params=pltpu.CompilerParams(use_tc_tiling_on_sc=False, needs_layout_passes=False),
    )


def _make_angle(tpad):
    nch = tpad // (NC * NS) // CH

    scratch = [
        pltpu.VMEM((nch, CH), jnp.int32),
        pltpu.VMEM((nch, CH), jnp.int32),
        pltpu.VMEM((CH, EMB), jnp.float32),
        pltpu.VMEM((CH, EMB), jnp.float32),
        pltpu.VMEM((CH, 4), jnp.float32),
    ]

    def body(emb, hidx, tidx, out, hb, tb, hrows, trows, dots):
        c = lax.axis_index("c")
        s = lax.axis_index("s")
        w = s * NC + c

        pltpu.sync_copy(hidx.at[pl.ds(w * nch, nch), :], hb)
        pltpu.sync_copy(tidx.at[pl.ds(w * nch, nch), :], tb)

        def txf(r, _):
            for k in range(CH // L):
                v = hb[r, pl.ds(k * L, L)]
                hb[r, pl.ds(k * L, L)] = v + NU
                v2 = tb[r, pl.ds(k * L, L)]
                tb[r, pl.ds(k * L, L)] = v2 + NU
            return 0
        lax.fori_loop(0, nch, txf, 0)

        lanes = lax.iota(jnp.int32, L)

        def do_chunk(j, _):
            pltpu.sync_copy(emb.at[hb.at[j]], hrows)
            pltpu.sync_copy(emb.at[tb.at[j]], trows)
            for g in range(CH // L):
                rowi = lanes + g * L
                zero = jnp.zeros((L,), jnp.float32)

                def dim_step(d, carry):
                    hh, tt, ht = carry
                    ci = jnp.full((L,), 0, jnp.int32) + d
                    h = plsc.load_gather(hrows, [rowi, ci])
                    t = plsc.load_gather(trows, [rowi, ci])
                    return (hh + h * h, tt + t * t, ht + h * t)

                hh, tt, ht = lax.fori_loop(0, EMB, dim_step, (zero, zero, zero))
                plsc.store_scatter(dots, [rowi, jnp.full((L,), 0, jnp.int32)], hh)
                plsc.store_scatter(dots, [rowi, jnp.full((L,), 1, jnp.int32)], tt)
                plsc.store_scatter(dots, [rowi, jnp.full((L,), 2, jnp.int32)], ht)
            pltpu.sync_copy(dots, out.at[pl.ds((w * nch + j) * CH, CH), :])
            return 0

        lax.fori_loop(0, nch, do_chunk, 0)

    return pl.kernel(
        body,
        out_type=jax.ShapeDtypeStruct((tpad, 4), jnp.float32),
        mesh=_mesh(),
        scratch_types=scratch,
        compiler_params=pltpu.CompilerParams(use_tc_tiling_on_sc=False, needs_layout_passes=False),
    )


def _pad_edges(x, fill):
    pad = EPAD - x.shape[0]
    if x.dtype == jnp.float32:
        tailv = jnp.full((pad,), fill, jnp.float32)
        return jnp.concatenate([x, tailv]).reshape(-1, CH)
    tailv = jnp.full((pad,), fill, jnp.int32)
    return jnp.concatenate([x.astype(jnp.int32), tailv]).reshape(-1, CH)


def kernel(user, pos_item, neg_item, all_embed, item_emb_cf, rel_emb,
           edge_index, edge_type, ui_rows, ui_cols, ui_vals, tri_head, tri_tail):
    f32 = jnp.float32
    head = edge_index[0].astype(jnp.int32)
    tail = edge_index[1].astype(jnp.int32)
    et = edge_type.astype(jnp.int32)

    tail_p = _pad_edges(tail, 0)
    head_p = _pad_edges(head, NENT)
    et_p = _pad_edges(et, 1)
    cols_src = _pad_edges(ui_cols.astype(jnp.int32), 0)
    cols_dst = _pad_edges(ui_cols.astype(jnp.int32), NI)
    rows_src = _pad_edges(ui_rows.astype(jnp.int32), 0)
    rows_dst = _pad_edges(ui_rows.astype(jnp.int32), NU)
    vals_i = lax.bitcast_convert_type(_pad_edges(ui_vals.astype(f32), 0.0),
                                      jnp.int32)

    kg_edges = jnp.stack([tail_p, head_p, et_p], axis=1)
    ua_edges = jnp.stack([cols_src, rows_dst, vals_i], axis=1)
    icf_edges = jnp.stack([rows_src, cols_dst, vals_i], axis=1)
    ones_i = lax.bitcast_convert_type(_pad_edges(jnp.ones((NEDGE,), f32), 0.0),
                                      jnp.int32)
    deg_edges = jnp.stack([jnp.zeros_like(tail_p), head_p, ones_i], axis=1)

    rel2 = rel_emb.astype(f32).reshape(4 * (NREL - 1), 16)
    all2 = all_embed.astype(f32).reshape(4 * (NU + NENT), 16)
    icf2 = item_emb_cf.astype(f32).reshape(4 * NI, 16)

    ones_tbl = jnp.ones((4, 16), f32)
    seg_deg = _make_segsum(4, ACC_E, RPT_E, "deg", 0)
    deg_full = seg_deg(ones_tbl, deg_edges)
    deg8 = deg_full[:, 0, 0:8]

    e_res = jnp.concatenate([all_embed[NU:], jnp.zeros((ACC_E - NENT, EMB), f32)])
    u_res = jnp.concatenate([all_embed[:NU], jnp.zeros((ACC_E - NU, EMB), f32)])
    i_res = jnp.concatenate([item_emb_cf, jnp.zeros((ACC_I - NI, EMB), f32)])

    seg_kg_0 = _make_segsum(4 * (NU + NENT), ACC_E, RPT_E, "rel", NU)
    seg_ua_0 = _make_segsum(4 * (NU + NENT), ACC_E, RPT_E, "val", NU)
    seg_kg = _make_segsum(4 * ACC_E, ACC_E, RPT_E, "rel", 0)
    seg_ua = _make_segsum(4 * ACC_E, ACC_E, RPT_E, "val", 0)
    seg_ucf_0 = _make_segsum(4 * NI, ACC_E, RPT_E, "val", 0)
    seg_ucf = _make_segsum(4 * ACC_I, ACC_E, RPT_E, "val", 0)
    seg_icf = _make_segsum(4 * ACC_E, ACC_I, RPT_I, "val", 0)

    ent2 = all2
    icf_t = icf2
    for hop in range(HOPS):
        if hop == 0:
            agg = seg_kg_0(ent2, kg_edges, rel2)
            uagg = seg_ua_0(ent2, ua_edges)
            ucf = seg_ucf_0(icf_t, ua_edges)
        else:
            agg = seg_kg(ent2, kg_edges, rel2)
            uagg = seg_ua(ent2, ua_edges)
            ucf = seg_ucf(icf_t, ua_edges)
        icf_new = seg_icf(ucf.reshape(4 * ACC_E, 16), icf_edges)

        ent_next, e_res = _div_norm_res(agg.reshape(ACC_E, EMB), deg8, e_res)
        u_res = _norm_res(uagg.reshape(ACC_E, EMB), u_res)
        i_res = _norm_res(icf_new.reshape(ACC_I, EMB), i_res)

        ent2 = ent_next.reshape(4 * ACC_E, 16)
        icf_t = icf_new.reshape(4 * ACC_I, 16)

    bg = _make_batch_gather()
    neg_flat = neg_item.reshape(-1).astype(jnp.int32)
    oue, ope, opc, one, onc = bg(
        u_res.reshape(2 * ACC_E, 32),
        e_res.reshape(2 * ACC_E, 32),
        i_res.reshape(2 * ACC_I, 32),
        user.astype(jnp.int32).reshape(-1, CH),
        pos_item.astype(jnp.int32).reshape(-1, CH),
        neg_flat.reshape(-1, CH))

    tot, su, sp, sne, spc, snc = _loss_partials(
        oue.reshape(BATCH, EMB),
        ope.reshape(BATCH, EMB),
        opc.reshape(BATCH, EMB),
        one.reshape(BATCH, NEG * EMB),
        onc.reshape(BATCH, NEG * EMB))
    loss1 = tot[0, 0] / BATCH
    reg = DECAY * (su + sp + sne + spc + snc)[0, 0] / (2.0 * BATCH)

    et_n = tri_head.shape[0]
    gran = NC * NS * CH
    tpad = max((et_n + gran - 1) // gran, 1) * gran
    th_p = jnp.concatenate(
        [tri_head.astype(jnp.int32),
         jnp.zeros((tpad - et_n,), jnp.int32)]).reshape(-1, CH)
    tt_p = jnp.concatenate(
        [tri_tail.astype(jnp.int32),
         jnp.zeros((tpad - et_n,), jnp.int32)]).reshape(-1, CH)
    ang = _make_angle(tpad)
    dots = ang(all_embed.astype(f32), th_p, tt_p)

    sc = ANGLE_DROP * ANGLE_DROP
    hh = dots[:et_n, 0] * sc
    tt = dots[:et_n, 1] * sc
    ht = dots[:et_n, 2] * sc
    eps = 1e-6
    nu_ = jnp.sqrt(hh)
    edist = jnp.sqrt(jnp.maximum(hh + tt - 2.0 * ht, 0.0))
    num = ht * (1.0 + hh) - hh * (1.0 + tt)
    denom = nu_ * edist * jnp.sqrt(jnp.clip(1.0 + tt * hh - 2.0 * ht, eps)) + eps
    angle = jnp.arccos(jnp.clip(num / denom, -1.0 + eps, 1.0 - eps))
    sqnu = jnp.clip(hh, 0.0, 1.0 - eps)
    half_ap = jnp.arcsin(jnp.clip(0.1 * (1.0 - sqnu) / jnp.sqrt(sqnu + eps),
                                  -1.0 + eps, 1.0 - eps))
    loss2 = ANGLE_W * jnp.sum(jnp.maximum(angle - half_ap, 0.0)) / et_n

    return loss1 + reg + loss2


# speedup vs baseline: 5.5194x; 1.3334x over previous
import jax
import jax.numpy as jnp
from jax import lax
from jax.experimental import pallas as pl
from jax.experimental.pallas import tpu as pltpu
from jax.experimental.pallas import tpu_sc as plsc

NU = 50000
NI = 20000
NENT = 50000
NREL = 17
EMB = 64
HOPS = 2
NEDGE = 800000
BATCH = 4096
NEG = 16
MARGIN = 0.8
DECAY = 1e-4
ANGLE_W = 0.5
ANGLE_DROP = 0.5

NC, NS, L = 2, 16, 16
CH = 128
SB = 56
NSB = 7
EPT = CH * SB * NSB
EPAD = EPT * NS

NR = EPAD // CH
NB = 8
ACC_E = 50176
RPT_E = ACC_E // NS
ACC_I = 20480
RPT_I = ACC_I // NS


def _mesh():
    return plsc.VectorSubcoreMesh(core_axis_name="c", subcore_axis_name="s")


def _make_segsum(tbl_rows4, accr, rpt, mode, off):
    scratch = [
        pltpu.VMEM((SB, 3, CH), jnp.int32),
        pltpu.VMEM((NB, CH, L), jnp.float32),
        pltpu.VMEM((112, L), jnp.float32),
        pltpu.VMEM_SHARED((accr, L), jnp.float32),
        pltpu.SemaphoreType.DMA((NB,)),
        pltpu.SemaphoreType.DMA((NB,)),
    ]
    if mode == "rel":
        scratch.insert(1, pltpu.VMEM((4 * (NREL - 1), L), jnp.float32))

    def body(tbl, edges, *rest):
        if mode == "rel":
            rel, out, esb, relv, rows, zbuf, acc, gsem, ssem = rest
        else:
            out, esb, rows, zbuf, acc, gsem, ssem = rest
            relv = None
        c = lax.axis_index("c")
        s = lax.axis_index("s")
        zv = jnp.zeros((L,), jnp.float32)
        for zi in range(112):
            zbuf[zi, pl.ds(0, L)] = zv
        if mode == "rel":
            pltpu.sync_copy(rel, relv)

        row0 = s * (SB * NSB)
        if mode == "deg":
            ov = jnp.full((L,), 1.0, jnp.float32)
            for zi in range(CH):
                rows[0, zi, pl.ds(0, L)] = ov

        for p in range(1 if mode == "deg" else 2):
            q = c + 2 * p

            def zloop(k, _):
                pltpu.sync_copy(zbuf, acc.at[pl.ds(s * rpt + k * 112, 112), :])
                return 0
            lax.fori_loop(0, rpt // 112, zloop, 0)
            plsc.subcore_barrier()

            def chunk(j, b):
                if mode != "deg":
                    pltpu.make_async_copy(tbl.at[esb.at[j, 0]], rows.at[b],
                                          gsem.at[b]).wait()
                if mode == "rel":
                    def grp(g, _):
                        i0 = g * L
                        tv = esb[j, 2, pl.ds(i0, L)]
                        for kk in range(L):
                            rr = 4 * tv[kk] + (q - 4)
                            relq = relv[rr, pl.ds(0, L)]
                            r_ = rows[b, i0 + kk, pl.ds(0, L)]
                            rows[b, i0 + kk, pl.ds(0, L)] = r_ * relq
                        return 0
                    lax.fori_loop(0, CH // L, grp, 0)
                elif mode == "val":
                    def grp(g, _):
                        i0 = g * L
                        vv = plsc.bitcast(esb[j, 2, pl.ds(i0, L)], jnp.float32)
                        for kk in range(L):
                            v = vv[kk]
                            r_ = rows[b, i0 + kk, pl.ds(0, L)]
                            rows[b, i0 + kk, pl.ds(0, L)] = r_ * v
                        return 0
                    lax.fori_loop(0, CH // L, grp, 0)
                if mode == "deg":
                    pltpu.sync_copy(rows.at[0], acc.at[esb.at[j, 1]], add=True)
                else:
                    pltpu.async_copy(rows.at[b], acc.at[esb.at[j, 1]],
                                     ssem.at[b], add=True)

            def super_chunk(sb, _):
                r0 = row0 + sb * SB
                pltpu.sync_copy(edges.at[pl.ds(r0, SB), :, :], esb)

                if mode != "deg":
                    def txf(r, _):
                        for k in range(CH // L):
                            v = esb[r, 0, pl.ds(k * L, L)]
                            esb[r, 0, pl.ds(k * L, L)] = v * 4 + (q + 4 * off)
                        return 0
                    lax.fori_loop(0, SB, txf, 0)

                    for t in range(NB):
                        pltpu.async_copy(tbl.at[esb.at[t, 0]], rows.at[t],
                                         gsem.at[t])

                def loop_body(m, _):
                    j0 = NB * m
                    for t in range(NB):
                        chunk(j0 + t, t)
                        if mode != "deg":
                            tp = (t + NB - 1) % NB
                            jr = j0 + t - 1 + NB

                            @pl.when((j0 + t - 1 >= 0) & (jr < SB))
                            def _(tp=tp, jr=jr):
                                pltpu.make_async_copy(
                                    rows.at[tp], acc.at[esb.at[0, 1]],
                                    ssem.at[tp]).wait()
                                pltpu.async_copy(tbl.at[esb.at[jr, 0]],
                                                 rows.at[tp], gsem.at[tp])
                    return 0

                lax.fori_loop(0, SB // NB, loop_body, 0)

                if mode != "deg":
                    def drain(t, _):
                        pltpu.make_async_copy(
                            rows.at[0], acc.at[esb.at[0, 1]],
                            ssem.at[t]).wait()
                        return 0
                    lax.fori_loop(0, NB, drain, 0)
                return 0

            lax.fori_loop(0, NSB, super_chunk, 0)
            plsc.subcore_barrier()

            pltpu.sync_copy(acc.at[pl.ds(s * rpt, rpt), :],
                            out.at[pl.ds(s * rpt, rpt), q, :])
            plsc.subcore_barrier()

    del tbl_rows4
    return pl.kernel(
        body,
        out_type=jax.ShapeDtypeStruct((accr, 4, L), jnp.float32),
        mesh=_mesh(),
        scratch_types=scratch,
        compiler_params=pltpu.CompilerParams(use_tc_tiling_on_sc=False,
                                             needs_layout_passes=False),
    )


def _norm_res(x, res):
    R = x.shape[0]
    blk = 512

    def body(x_ref, r_ref, o_ref):
        xv = x_ref[...]
        nrm = jnp.sqrt(jnp.sum(xv * xv, axis=1, keepdims=True))
        o_ref[...] = r_ref[...] + xv / (nrm + 1e-8)

    return pl.pallas_call(
        body,
        out_shape=jax.ShapeDtypeStruct((R, EMB), jnp.float32),
        grid=(R // blk,),
        in_specs=[pl.BlockSpec((blk, EMB), lambda i: (i, 0)),
                  pl.BlockSpec((blk, EMB), lambda i: (i, 0))],
        out_specs=pl.BlockSpec((blk, EMB), lambda i: (i, 0)),
    )(x, res)


def _div_norm_res(x, deg8, res):
    R = x.shape[0]
    blk = 512

    def body(x_ref, d_ref, r_ref, e_ref, o_ref):
        xv = x_ref[...]
        d = jnp.maximum(d_ref[:, 0:1], 1.0)
        e_ref[...] = xv / d
        nrm = jnp.sqrt(jnp.sum(xv * xv, axis=1, keepdims=True))
        o_ref[...] = r_ref[...] + xv / (nrm + 1e-8)

    return pl.pallas_call(
        body,
        out_shape=(jax.ShapeDtypeStruct((R, EMB), jnp.float32),
                   jax.ShapeDtypeStruct((R, EMB), jnp.float32)),
        grid=(R // blk,),
        in_specs=[pl.BlockSpec((blk, EMB), lambda i: (i, 0)),
                  pl.BlockSpec((blk, 8), lambda i: (i, 0)),
                  pl.BlockSpec((blk, EMB), lambda i: (i, 0))],
        out_specs=(pl.BlockSpec((blk, EMB), lambda i: (i, 0)),
                   pl.BlockSpec((blk, EMB), lambda i: (i, 0))),
    )(x, deg8, res)


def _loss_partials(ue, pe, pc, ne2, nc2):
    blk = 512
    G = BATCH // blk

    def body(u_ref, p_ref, pc_ref, n_ref, nc_ref, o_ref, *o_rest):
        u_raw = u_ref[...]
        p_raw = p_ref[...]
        pc_raw = pc_ref[...]
        u = u_raw / (jnp.sqrt(jnp.sum(u_raw * u_raw, axis=1, keepdims=True)) + 1e-8)
        ps = p_raw + pc_raw
        p = ps / (jnp.sqrt(jnp.sum(ps * ps, axis=1, keepdims=True)) + 1e-8)
        pos_score = jnp.sum(u * p, axis=1, keepdims=True)
        hinge = jnp.maximum(1.0 - pos_score, 0.0)
        neg_acc = jnp.zeros((blk, 1), jnp.float32)
        sq_ne = jnp.zeros((), jnp.float32)
        sq_nc = jnp.zeros((), jnp.float32)
        for k in range(NEG):
            nk_raw = n_ref[:, k * EMB:(k + 1) * EMB]
            nck_raw = nc_ref[:, k * EMB:(k + 1) * EMB]
            sq_ne += jnp.sum(nk_raw * nk_raw)
            sq_nc += jnp.sum(nck_raw * nck_raw)
            nk = nk_raw + nck_raw
            nkn = nk / (jnp.sqrt(jnp.sum(nk * nk, axis=1, keepdims=True)) + 1e-8)
            ns = jnp.sum(u * nkn, axis=1, keepdims=True)
            neg_acc += jnp.maximum(ns - MARGIN, 0.0)
        tot = jnp.sum(hinge + neg_acc * (1.0 / NEG))
        cols = [tot,
                jnp.sum(u_raw * u_raw),
                jnp.sum(p_raw * p_raw),
                sq_ne,
                jnp.sum(pc_raw * pc_raw),
                sq_nc]
        orefs = [o_ref] + list(o_rest)
        pid = pl.program_id(0)
        for oref, v in zip(orefs, cols):
            @pl.when(pid == 0)
            def _(oref=oref):
                oref[...] = jnp.zeros((1, 1), jnp.float32)
            oref[...] += v.reshape(1, 1)

    scal = jax.ShapeDtypeStruct((1, 1), jnp.float32)
    sspec = pl.BlockSpec((1, 1), lambda i: (0, 0))
    return pl.pallas_call(
        body,
        out_shape=(scal,) * 6,
        grid=(G,),
        in_specs=[pl.BlockSpec((blk, EMB), lambda i: (i, 0)),
                  pl.BlockSpec((blk, EMB), lambda i: (i, 0)),
                  pl.BlockSpec((blk, EMB), lambda i: (i, 0)),
                  pl.BlockSpec((blk, NEG * EMB), lambda i: (i, 0)),
                  pl.BlockSpec((blk, NEG * EMB), lambda i: (i, 0))],
        out_specs=(sspec,) * 6,
    )(ue, pe, pc, ne2, nc2)


def _make_batch_gather():
    nch_u = BATCH // NS // CH
    nch_n = BATCH * NEG // NS // CH

    scratch = [
        pltpu.VMEM((nch_u, CH), jnp.int32),
        pltpu.VMEM((nch_u, CH), jnp.int32),
        pltpu.VMEM((nch_n, CH), jnp.int32),
        pltpu.VMEM((CH, 32), jnp.float32),
    ]
    out_t = (
        jax.ShapeDtypeStruct((BATCH, 2, 32), jnp.float32),
        jax.ShapeDtypeStruct((BATCH, 2, 32), jnp.float32),
        jax.ShapeDtypeStruct((BATCH, 2, 32), jnp.float32),
        jax.ShapeDtypeStruct((BATCH * NEG, 2, 32), jnp.float32),
        jax.ShapeDtypeStruct((BATCH * NEG, 2, 32), jnp.float32),
    )

    def body(ures, eres, ires, uidx, pidx, nidx,
             oue, ope, opc, one, onc, ub, pb, nb, rbuf):
        c = lax.axis_index("c")
        s = lax.axis_index("s")

        def load_transform(src_hbm, buf, nch):
            pltpu.sync_copy(src_hbm.at[pl.ds(s * nch, nch), :], buf)
            for r in range(nch):
                for k in range(CH // L):
                    v = buf[r, pl.ds(k * L, L)]
                    buf[r, pl.ds(k * L, L)] = v * 2 + c

        load_transform(uidx, ub, nch_u)
        load_transform(pidx, pb, nch_u)
        load_transform(nidx, nb, nch_n)

        def job(tbl, buf, nch, out):
            for j in range(nch):
                pltpu.sync_copy(tbl.at[buf.at[j]], rbuf)
                row = (s * nch + j) * CH

                @pl.when(c == 0)
                def _():
                    pltpu.sync_copy(rbuf, out.at[pl.ds(row, CH), 0, :])

                @pl.when(c == 1)
                def _():
                    pltpu.sync_copy(rbuf, out.at[pl.ds(row, CH), 1, :])

        job(ures, ub, nch_u, oue)
        job(eres, pb, nch_u, ope)
        job(ires, pb, nch_u, opc)
        job(eres, nb, nch_n, one)
        job(ires, nb, nch_n, onc)

    return pl.kernel(
        body,
        out_type=out_t,
        mesh=_mesh(),
        scratch_types=scratch,
        compiler_params=pltpu.CompilerParams(use_tc_tiling_on_sc=False, needs_layout_passes=False),
    )


def _make_angle(tpad):
    nch = tpad // (NC * NS) // CH

    scratch = [
        pltpu.VMEM((nch, CH), jnp.int32),
        pltpu.VMEM((nch, CH), jnp.int32),
        pltpu.VMEM((CH, EMB), jnp.float32),
        pltpu.VMEM((CH, EMB), jnp.float32),
        pltpu.VMEM((CH, 4), jnp.float32),
    ]

    def body(emb, hidx, tidx, out, hb, tb, hrows, trows, dots):
        c = lax.axis_index("c")
        s = lax.axis_index("s")
        w = s * NC + c

        pltpu.sync_copy(hidx.at[pl.ds(w * nch, nch), :], hb)
        pltpu.sync_copy(tidx.at[pl.ds(w * nch, nch), :], tb)

        def txf(r, _):
            for k in range(CH // L):
                v = hb[r, pl.ds(k * L, L)]
                hb[r, pl.ds(k * L, L)] = v + NU
                v2 = tb[r, pl.ds(k * L, L)]
                tb[r, pl.ds(k * L, L)] = v2 + NU
            return 0
        lax.fori_loop(0, nch, txf, 0)

        lanes = lax.iota(jnp.int32, L)

        def do_chunk(j, _):
            pltpu.sync_copy(emb.at[hb.at[j]], hrows)
            pltpu.sync_copy(emb.at[tb.at[j]], trows)
            for g in range(CH // L):
                rowi = lanes + g * L
                zero = jnp.zeros((L,), jnp.float32)

                def dim_step(d, carry):
                    hh, tt, ht = carry
                    ci = jnp.full((L,), 0, jnp.int32) + d
                    h = plsc.load_gather(hrows, [rowi, ci])
                    t = plsc.load_gather(trows, [rowi, ci])
                    return (hh + h * h, tt + t * t, ht + h * t)

                hh, tt, ht = lax.fori_loop(0, EMB, dim_step, (zero, zero, zero))
                plsc.store_scatter(dots, [rowi, jnp.full((L,), 0, jnp.int32)], hh)
                plsc.store_scatter(dots, [rowi, jnp.full((L,), 1, jnp.int32)], tt)
                plsc.store_scatter(dots, [rowi, jnp.full((L,), 2, jnp.int32)], ht)
            pltpu.sync_copy(dots, out.at[pl.ds((w * nch + j) * CH, CH), :])
            return 0

        lax.fori_loop(0, nch, do_chunk, 0)

    return pl.kernel(
        body,
        out_type=jax.ShapeDtypeStruct((tpad, 4), jnp.float32),
        mesh=_mesh(),
        scratch_types=scratch,
        compiler_params=pltpu.CompilerParams(use_tc_tiling_on_sc=False, needs_layout_passes=False),
    )


def _pad_edges(x, fill):
    pad = EPAD - x.shape[0]
    if x.dtype == jnp.float32:
        tailv = jnp.full((pad,), fill, jnp.float32)
        return jnp.concatenate([x, tailv]).reshape(-1, CH)
    tailv = jnp.full((pad,), fill, jnp.int32)
    return jnp.concatenate([x.astype(jnp.int32), tailv]).reshape(-1, CH)


def kernel(user, pos_item, neg_item, all_embed, item_emb_cf, rel_emb,
           edge_index, edge_type, ui_rows, ui_cols, ui_vals, tri_head, tri_tail):
    f32 = jnp.float32
    head = edge_index[0].astype(jnp.int32)
    tail = edge_index[1].astype(jnp.int32)
    et = edge_type.astype(jnp.int32)

    tail_p = _pad_edges(tail, 0)
    head_p = _pad_edges(head, NENT)
    et_p = _pad_edges(et, 1)
    cols_src = _pad_edges(ui_cols.astype(jnp.int32), 0)
    cols_dst = _pad_edges(ui_cols.astype(jnp.int32), NI)
    rows_src = _pad_edges(ui_rows.astype(jnp.int32), 0)
    rows_dst = _pad_edges(ui_rows.astype(jnp.int32), NU)
    vals_i = lax.bitcast_convert_type(_pad_edges(ui_vals.astype(f32), 0.0),
                                      jnp.int32)

    kg_edges = jnp.stack([tail_p, head_p, et_p], axis=1)
    ua_edges = jnp.stack([cols_src, rows_dst, vals_i], axis=1)
    icf_edges = jnp.stack([rows_src, cols_dst, vals_i], axis=1)
    ones_i = lax.bitcast_convert_type(_pad_edges(jnp.ones((NEDGE,), f32), 0.0),
                                      jnp.int32)
    deg_edges = jnp.stack([jnp.zeros_like(tail_p), head_p, ones_i], axis=1)

    rel2 = rel_emb.astype(f32).reshape(4 * (NREL - 1), 16)
    all2 = all_embed.astype(f32).reshape(4 * (NU + NENT), 16)
    icf2 = item_emb_cf.astype(f32).reshape(4 * NI, 16)

    ones_tbl = jnp.ones((4, 16), f32)
    seg_deg = _make_segsum(4, ACC_E, RPT_E, "deg", 0)
    deg_full = seg_deg(ones_tbl, deg_edges)
    deg8 = deg_full[:, 0, 0:8]

    e_res = jnp.concatenate([all_embed[NU:], jnp.zeros((ACC_E - NENT, EMB), f32)])
    u_res = jnp.concatenate([all_embed[:NU], jnp.zeros((ACC_E - NU, EMB), f32)])
    i_res = jnp.concatenate([item_emb_cf, jnp.zeros((ACC_I - NI, EMB), f32)])

    seg_kg_0 = _make_segsum(4 * (NU + NENT), ACC_E, RPT_E, "rel", NU)
    seg_ua_0 = _make_segsum(4 * (NU + NENT), ACC_E, RPT_E, "val", NU)
    seg_kg = _make_segsum(4 * ACC_E, ACC_E, RPT_E, "rel", 0)
    seg_ua = _make_segsum(4 * ACC_E, ACC_E, RPT_E, "val", 0)
    seg_ucf_0 = _make_segsum(4 * NI, ACC_E, RPT_E, "val", 0)
    seg_ucf = _make_segsum(4 * ACC_I, ACC_E, RPT_E, "val", 0)
    seg_icf = _make_segsum(4 * ACC_E, ACC_I, RPT_I, "val", 0)

    ent2 = all2
    icf_t = icf2
    for hop in range(HOPS):
        if hop == 0:
            agg = seg_kg_0(ent2, kg_edges, rel2)
            uagg = seg_ua_0(ent2, ua_edges)
            ucf = seg_ucf_0(icf_t, ua_edges)
        else:
            agg = seg_kg(ent2, kg_edges, rel2)
            uagg = seg_ua(ent2, ua_edges)
            ucf = seg_ucf(icf_t, ua_edges)
        icf_new = seg_icf(ucf.reshape(4 * ACC_E, 16), icf_edges)

        ent_next, e_res = _div_norm_res(agg.reshape(ACC_E, EMB), deg8, e_res)
        u_res = _norm_res(uagg.reshape(ACC_E, EMB), u_res)
        i_res = _norm_res(icf_new.reshape(ACC_I, EMB), i_res)

        ent2 = ent_next.reshape(4 * ACC_E, 16)
        icf_t = icf_new.reshape(4 * ACC_I, 16)

    bg = _make_batch_gather()
    neg_flat = neg_item.reshape(-1).astype(jnp.int32)
    oue, ope, opc, one, onc = bg(
        u_res.reshape(2 * ACC_E, 32),
        e_res.reshape(2 * ACC_E, 32),
        i_res.reshape(2 * ACC_I, 32),
        user.astype(jnp.int32).reshape(-1, CH),
        pos_item.astype(jnp.int32).reshape(-1, CH),
        neg_flat.reshape(-1, CH))

    tot, su, sp, sne, spc, snc = _loss_partials(
        oue.reshape(BATCH, EMB),
        ope.reshape(BATCH, EMB),
        opc.reshape(BATCH, EMB),
        one.reshape(BATCH, NEG * EMB),
        onc.reshape(BATCH, NEG * EMB))
    loss1 = tot[0, 0] / BATCH
    reg = DECAY * (su + sp + sne + spc + snc)[0, 0] / (2.0 * BATCH)

    et_n = tri_head.shape[0]
    gran = NC * NS * CH
    tpad = max((et_n + gran - 1) // gran, 1) * gran
    th_p = jnp.concatenate(
        [tri_head.astype(jnp.int32),
         jnp.zeros((tpad - et_n,), jnp.int32)]).reshape(-1, CH)
    tt_p = jnp.concatenate(
        [tri_tail.astype(jnp.int32),
         jnp.zeros((tpad - et_n,), jnp.int32)]).reshape(-1, CH)
    ang = _make_angle(tpad)
    dots = ang(all_embed.astype(f32), th_p, tt_p)

    sc = ANGLE_DROP * ANGLE_DROP
    hh = dots[:et_n, 0] * sc
    tt = dots[:et_n, 1] * sc
    ht = dots[:et_n, 2] * sc
    eps = 1e-6
    nu_ = jnp.sqrt(hh)
    edist = jnp.sqrt(jnp.maximum(hh + tt - 2.0 * ht, 0.0))
    num = ht * (1.0 + hh) - hh * (1.0 + tt)
    denom = nu_ * edist * jnp.sqrt(jnp.clip(1.0 + tt * hh - 2.0 * ht, eps)) + eps
    angle = jnp.arccos(jnp.clip(num / denom, -1.0 + eps, 1.0 - eps))
    sqnu = jnp.clip(hh, 0.0, 1.0 - eps)
    half_ap = jnp.arcsin(jnp.clip(0.1 * (1.0 - sqnu) / jnp.sqrt(sqnu + eps),
                                  -1.0 + eps, 1.0 - eps))
    loss2 = ANGLE_W * jnp.sum(jnp.maximum(angle - half_ap, 0.0)) / et_n

    return loss1 + reg + loss2


# speedup vs baseline: 5.7779x; 1.0468x over previous
import jax
import jax.numpy as jnp
from jax import lax
from jax.experimental import pallas as pl
from jax.experimental.pallas import tpu as pltpu
from jax.experimental.pallas import tpu_sc as plsc

NU = 50000
NI = 20000
NENT = 50000
NREL = 17
EMB = 64
HOPS = 2
NEDGE = 800000
BATCH = 4096
NEG = 16
MARGIN = 0.8
DECAY = 1e-4
ANGLE_W = 0.5
ANGLE_DROP = 0.5

NC, NS, L = 2, 16, 16
CH = 128
SB = 56
NSB = 7
EPT = CH * SB * NSB
EPAD = EPT * NS

NR = EPAD // CH
NB = 8
ACC_E = 50176
RPT_E = ACC_E // NS
ACC_I = 20480
RPT_I = ACC_I // NS


def _mesh():
    return plsc.VectorSubcoreMesh(core_axis_name="c", subcore_axis_name="s")


def _make_segsum(tbl_rows4, accr, rpt, mode, off):
    scratch = [
        pltpu.VMEM((SB, 3, CH), jnp.int32),
        pltpu.VMEM((NB, CH, L), jnp.float32),
        pltpu.VMEM((112, L), jnp.float32),
        pltpu.VMEM_SHARED((accr, L), jnp.float32),
        pltpu.SemaphoreType.DMA((NB,)),
        pltpu.SemaphoreType.DMA((NB,)),
    ]
    if mode == "rel":
        scratch.insert(1, pltpu.VMEM((4 * (NREL - 1), L), jnp.float32))

    def body(tbl, edges, *rest):
        if mode == "rel":
            rel, out, esb, relv, rows, zbuf, acc, gsem, ssem = rest
        else:
            out, esb, rows, zbuf, acc, gsem, ssem = rest
            relv = None
        c = lax.axis_index("c")
        s = lax.axis_index("s")
        zv = jnp.zeros((L,), jnp.float32)
        for zi in range(112):
            zbuf[zi, pl.ds(0, L)] = zv
        if mode == "rel":
            pltpu.sync_copy(rel, relv)

        row0 = s * (SB * NSB)
        if mode == "deg":
            ov = jnp.full((L,), 1.0, jnp.float32)
            for zi in range(CH):
                rows[0, zi, pl.ds(0, L)] = ov

        for p in range(1 if mode == "deg" else 2):
            q = c + 2 * p

            def zloop(k, _):
                pltpu.sync_copy(zbuf, acc.at[pl.ds(s * rpt + k * 112, 112), :])
                return 0
            lax.fori_loop(0, rpt // 112, zloop, 0)
            plsc.subcore_barrier()

            def chunk(j, b):
                if mode != "deg":
                    pltpu.make_async_copy(tbl.at[esb.at[j, 0]], rows.at[b],
                                          gsem.at[b]).wait()
                if mode == "rel":
                    def grp(g, _):
                        i0 = g * L
                        tv = esb[j, 2, pl.ds(i0, L)]
                        for kk in range(L):
                            rr = 4 * tv[kk] + (q - 4)
                            relq = relv[rr, pl.ds(0, L)]
                            r_ = rows[b, i0 + kk, pl.ds(0, L)]
                            rows[b, i0 + kk, pl.ds(0, L)] = r_ * relq
                        return 0
                    lax.fori_loop(0, CH // L, grp, 0)
                elif mode == "val":
                    def grp(g, _):
                        i0 = g * L
                        vv = plsc.bitcast(esb[j, 2, pl.ds(i0, L)], jnp.float32)
                        for kk in range(L):
                            v = vv[kk]
                            r_ = rows[b, i0 + kk, pl.ds(0, L)]
                            rows[b, i0 + kk, pl.ds(0, L)] = r_ * v
                        return 0
                    lax.fori_loop(0, CH // L, grp, 0)
                if mode == "deg":
                    pltpu.sync_copy(rows.at[0], acc.at[esb.at[j, 1]], add=True)
                else:
                    pltpu.async_copy(rows.at[b], acc.at[esb.at[j, 1]],
                                     ssem.at[b], add=True)

            def super_chunk(sb, _):
                r0 = row0 + sb * SB
                pltpu.sync_copy(edges.at[pl.ds(r0, SB), :, :], esb)

                if mode != "deg":
                    def txf(r, _):
                        for k in range(CH // L):
                            v = esb[r, 0, pl.ds(k * L, L)]
                            esb[r, 0, pl.ds(k * L, L)] = v * 4 + (q + 4 * off)
                        return 0
                    lax.fori_loop(0, SB, txf, 0)

                    for t in range(NB):
                        pltpu.async_copy(tbl.at[esb.at[t, 0]], rows.at[t],
                                         gsem.at[t])

                def loop_body(m, _):
                    j0 = NB * m
                    for t in range(NB):
                        chunk(j0 + t, t)
                        if mode != "deg":
                            tp = (t + NB - 1) % NB
                            jr = j0 + t - 1 + NB

                            @pl.when((j0 + t - 1 >= 0) & (jr < SB))
                            def _(tp=tp, jr=jr):
                                pltpu.make_async_copy(
                                    rows.at[tp], acc.at[esb.at[0, 1]],
                                    ssem.at[tp]).wait()
                                pltpu.async_copy(tbl.at[esb.at[jr, 0]],
                                                 rows.at[tp], gsem.at[tp])
                    return 0

                lax.fori_loop(0, SB // NB, loop_body, 0)

                if mode != "deg":
                    def drain(t, _):
                        pltpu.make_async_copy(
                            rows.at[0], acc.at[esb.at[0, 1]],
                            ssem.at[t]).wait()
                        return 0
                    lax.fori_loop(0, NB, drain, 0)
                return 0

            lax.fori_loop(0, NSB, super_chunk, 0)
            plsc.subcore_barrier()

            pltpu.sync_copy(acc.at[pl.ds(s * rpt, rpt), :],
                            out.at[pl.ds(s * rpt, rpt), q, :])
            plsc.subcore_barrier()

    del tbl_rows4
    return pl.kernel(
        body,
        out_type=jax.ShapeDtypeStruct((accr, 4, L), jnp.float32),
        mesh=_mesh(),
        scratch_types=scratch,
        compiler_params=pltpu.CompilerParams(use_tc_tiling_on_sc=False,
                                             needs_layout_passes=False),
    )


def _norm_res(x, res):
    R = x.shape[0]
    blk = 512

    def body(x_ref, r_ref, o_ref):
        xv = x_ref[...]
        nrm = jnp.sqrt(jnp.sum(xv * xv, axis=1, keepdims=True))
        o_ref[...] = r_ref[...] + xv / (nrm + 1e-8)

    return pl.pallas_call(
        body,
        out_shape=jax.ShapeDtypeStruct((R, EMB), jnp.float32),
        grid=(R // blk,),
        in_specs=[pl.BlockSpec((blk, EMB), lambda i: (i, 0)),
                  pl.BlockSpec((blk, EMB), lambda i: (i, 0))],
        out_specs=pl.BlockSpec((blk, EMB), lambda i: (i, 0)),
    )(x, res)


def _div_norm_res(x, deg8, res):
    R = x.shape[0]
    blk = 512

    def body(x_ref, d_ref, r_ref, e_ref, o_ref):
        xv = x_ref[...]
        d = jnp.maximum(d_ref[:, 0:1], 1.0)
        e_ref[...] = xv / d
        nrm = jnp.sqrt(jnp.sum(xv * xv, axis=1, keepdims=True))
        o_ref[...] = r_ref[...] + xv / (nrm + 1e-8)

    return pl.pallas_call(
        body,
        out_shape=(jax.ShapeDtypeStruct((R, EMB), jnp.float32),
                   jax.ShapeDtypeStruct((R, EMB), jnp.float32)),
        grid=(R // blk,),
        in_specs=[pl.BlockSpec((blk, EMB), lambda i: (i, 0)),
                  pl.BlockSpec((blk, 8), lambda i: (i, 0)),
                  pl.BlockSpec((blk, EMB), lambda i: (i, 0))],
        out_specs=(pl.BlockSpec((blk, EMB), lambda i: (i, 0)),
                   pl.BlockSpec((blk, EMB), lambda i: (i, 0))),
    )(x, deg8, res)


def _loss_partials(ue, pe, pc, ne2, nc2):
    blk = 512
    G = BATCH // blk

    def body(u_ref, p_ref, pc_ref, n_ref, nc_ref, o_ref, *o_rest):
        u_raw = u_ref[...]
        p_raw = p_ref[...]
        pc_raw = pc_ref[...]
        u = u_raw / (jnp.sqrt(jnp.sum(u_raw * u_raw, axis=1, keepdims=True)) + 1e-8)
        ps = p_raw + pc_raw
        p = ps / (jnp.sqrt(jnp.sum(ps * ps, axis=1, keepdims=True)) + 1e-8)
        pos_score = jnp.sum(u * p, axis=1, keepdims=True)
        hinge = jnp.maximum(1.0 - pos_score, 0.0)
        neg_acc = jnp.zeros((blk, 1), jnp.float32)
        sq_ne = jnp.zeros((), jnp.float32)
        sq_nc = jnp.zeros((), jnp.float32)
        for k in range(NEG):
            nk_raw = n_ref[:, k * EMB:(k + 1) * EMB]
            nck_raw = nc_ref[:, k * EMB:(k + 1) * EMB]
            sq_ne += jnp.sum(nk_raw * nk_raw)
            sq_nc += jnp.sum(nck_raw * nck_raw)
            nk = nk_raw + nck_raw
            nkn = nk / (jnp.sqrt(jnp.sum(nk * nk, axis=1, keepdims=True)) + 1e-8)
            ns = jnp.sum(u * nkn, axis=1, keepdims=True)
            neg_acc += jnp.maximum(ns - MARGIN, 0.0)
        tot = jnp.sum(hinge + neg_acc * (1.0 / NEG))
        cols = [tot,
                jnp.sum(u_raw * u_raw),
                jnp.sum(p_raw * p_raw),
                sq_ne,
                jnp.sum(pc_raw * pc_raw),
                sq_nc]
        orefs = [o_ref] + list(o_rest)
        pid = pl.program_id(0)
        for oref, v in zip(orefs, cols):
            @pl.when(pid == 0)
            def _(oref=oref):
                oref[...] = jnp.zeros((1, 1), jnp.float32)
            oref[...] += v.reshape(1, 1)

    scal = jax.ShapeDtypeStruct((1, 1), jnp.float32)
    sspec = pl.BlockSpec((1, 1), lambda i: (0, 0))
    return pl.pallas_call(
        body,
        out_shape=(scal,) * 6,
        grid=(G,),
        in_specs=[pl.BlockSpec((blk, EMB), lambda i: (i, 0)),
                  pl.BlockSpec((blk, EMB), lambda i: (i, 0)),
                  pl.BlockSpec((blk, EMB), lambda i: (i, 0)),
                  pl.BlockSpec((blk, NEG * EMB), lambda i: (i, 0)),
                  pl.BlockSpec((blk, NEG * EMB), lambda i: (i, 0))],
        out_specs=(sspec,) * 6,
    )(ue, pe, pc, ne2, nc2)


def _make_batch_gather():
    nch_u = BATCH // NS // CH
    nch_n = BATCH * NEG // NS // CH

    scratch = [
        pltpu.VMEM((nch_u, CH), jnp.int32),
        pltpu.VMEM((nch_u, CH), jnp.int32),
        pltpu.VMEM((nch_n, CH), jnp.int32),
        pltpu.VMEM((CH, 32), jnp.float32),
    ]
    out_t = (
        jax.ShapeDtypeStruct((BATCH, 2, 32), jnp.float32),
        jax.ShapeDtypeStruct((BATCH, 2, 32), jnp.float32),
        jax.ShapeDtypeStruct((BATCH, 2, 32), jnp.float32),
        jax.ShapeDtypeStruct((BATCH * NEG, 2, 32), jnp.float32),
        jax.ShapeDtypeStruct((BATCH * NEG, 2, 32), jnp.float32),
    )

    def body(ures, eres, ires, uidx, pidx, nidx,
             oue, ope, opc, one, onc, ub, pb, nb, rbuf):
        c = lax.axis_index("c")
        s = lax.axis_index("s")

        def load_transform(src_hbm, buf, nch):
            pltpu.sync_copy(src_hbm.at[pl.ds(s * nch, nch), :], buf)
            for r in range(nch):
                for k in range(CH // L):
                    v = buf[r, pl.ds(k * L, L)]
                    buf[r, pl.ds(k * L, L)] = v * 2 + c

        load_transform(uidx, ub, nch_u)
        load_transform(pidx, pb, nch_u)
        load_transform(nidx, nb, nch_n)

        def job(tbl, buf, nch, out):
            for j in range(nch):
                pltpu.sync_copy(tbl.at[buf.at[j]], rbuf)
                row = (s * nch + j) * CH

                @pl.when(c == 0)
                def _():
                    pltpu.sync_copy(rbuf, out.at[pl.ds(row, CH), 0, :])

                @pl.when(c == 1)
                def _():
                    pltpu.sync_copy(rbuf, out.at[pl.ds(row, CH), 1, :])

        job(ures, ub, nch_u, oue)
        job(eres, pb, nch_u, ope)
        job(ires, pb, nch_u, opc)
        job(eres, nb, nch_n, one)
        job(ires, nb, nch_n, onc)

    return pl.kernel(
        body,
        out_type=out_t,
        mesh=_mesh(),
        scratch_types=scratch,
        compiler_params=pltpu.CompilerParams(use_tc_tiling_on_sc=False, needs_layout_passes=False),
    )


def _make_angle(tpad):
    nch = tpad // (NC * NS) // CH

    scratch = [
        pltpu.VMEM((nch, CH), jnp.int32),
        pltpu.VMEM((nch, CH), jnp.int32),
        pltpu.VMEM((2, CH, EMB), jnp.float32),
        pltpu.VMEM((2, CH, EMB), jnp.float32),
        pltpu.VMEM((CH, 4), jnp.float32),
        pltpu.SemaphoreType.DMA((2,)),
        pltpu.SemaphoreType.DMA((2,)),
    ]

    def body(emb, hidx, tidx, out, hb, tb, hrows, trows, dots, hsem, tsem):
        c = lax.axis_index("c")
        s = lax.axis_index("s")
        w = s * NC + c

        pltpu.sync_copy(hidx.at[pl.ds(w * nch, nch), :], hb)
        pltpu.sync_copy(tidx.at[pl.ds(w * nch, nch), :], tb)

        def txf(r, _):
            for k in range(CH // L):
                v = hb[r, pl.ds(k * L, L)]
                hb[r, pl.ds(k * L, L)] = v + NU
                v2 = tb[r, pl.ds(k * L, L)]
                tb[r, pl.ds(k * L, L)] = v2 + NU
            return 0
        lax.fori_loop(0, nch, txf, 0)

        lanes = lax.iota(jnp.int32, L)

        for bb in range(2):
            pltpu.async_copy(emb.at[hb.at[bb]], hrows.at[bb], hsem.at[bb])
            pltpu.async_copy(emb.at[tb.at[bb]], trows.at[bb], tsem.at[bb])

        def do_chunk(j, bb):
            pltpu.make_async_copy(emb.at[hb.at[j]], hrows.at[bb],
                                  hsem.at[bb]).wait()
            pltpu.make_async_copy(emb.at[tb.at[j]], trows.at[bb],
                                  tsem.at[bb]).wait()
            for g in range(CH // L):
                rowi = lanes + g * L
                zero = jnp.zeros((L,), jnp.float32)

                def dim_step(d, carry):
                    hh, tt, ht = carry
                    ci = jnp.full((L,), 0, jnp.int32) + d
                    h = plsc.load_gather(hrows.at[bb], [rowi, ci])
                    t = plsc.load_gather(trows.at[bb], [rowi, ci])
                    return (hh + h * h, tt + t * t, ht + h * t)

                hh, tt, ht = lax.fori_loop(0, EMB, dim_step, (zero, zero, zero))
                plsc.store_scatter(dots, [rowi, jnp.full((L,), 0, jnp.int32)], hh)
                plsc.store_scatter(dots, [rowi, jnp.full((L,), 1, jnp.int32)], tt)
                plsc.store_scatter(dots, [rowi, jnp.full((L,), 2, jnp.int32)], ht)
            pltpu.sync_copy(dots, out.at[pl.ds((w * nch + j) * CH, CH), :])

        def loop_body(m, _):
            j0 = 2 * m
            for bb in range(2):
                do_chunk(j0 + bb, bb)

                @pl.when(j0 + bb + 2 < nch)
                def _(bb=bb):
                    pltpu.async_copy(emb.at[hb.at[j0 + bb + 2]], hrows.at[bb],
                                     hsem.at[bb])
                    pltpu.async_copy(emb.at[tb.at[j0 + bb + 2]], trows.at[bb],
                                     tsem.at[bb])
            return 0

        lax.fori_loop(0, nch // 2, loop_body, 0)

    return pl.kernel(
        body,
        out_type=jax.ShapeDtypeStruct((tpad, 4), jnp.float32),
        mesh=_mesh(),
        scratch_types=scratch,
        compiler_params=pltpu.CompilerParams(use_tc_tiling_on_sc=False, needs_layout_passes=False),
    )


def _pad_edges(x, fill):
    pad = EPAD - x.shape[0]
    if x.dtype == jnp.float32:
        tailv = jnp.full((pad,), fill, jnp.float32)
        return jnp.concatenate([x, tailv]).reshape(-1, CH)
    tailv = jnp.full((pad,), fill, jnp.int32)
    return jnp.concatenate([x.astype(jnp.int32), tailv]).reshape(-1, CH)


def kernel(user, pos_item, neg_item, all_embed, item_emb_cf, rel_emb,
           edge_index, edge_type, ui_rows, ui_cols, ui_vals, tri_head, tri_tail):
    f32 = jnp.float32
    head = edge_index[0].astype(jnp.int32)
    tail = edge_index[1].astype(jnp.int32)
    et = edge_type.astype(jnp.int32)

    tail_p = _pad_edges(tail, 0)
    head_p = _pad_edges(head, NENT)
    et_p = _pad_edges(et, 1)
    cols_src = _pad_edges(ui_cols.astype(jnp.int32), 0)
    cols_dst = _pad_edges(ui_cols.astype(jnp.int32), NI)
    rows_src = _pad_edges(ui_rows.astype(jnp.int32), 0)
    rows_dst = _pad_edges(ui_rows.astype(jnp.int32), NU)
    vals_i = lax.bitcast_convert_type(_pad_edges(ui_vals.astype(f32), 0.0),
                                      jnp.int32)

    kg_edges = jnp.stack([tail_p, head_p, et_p], axis=1)
    ua_edges = jnp.stack([cols_src, rows_dst, vals_i], axis=1)
    icf_edges = jnp.stack([rows_src, cols_dst, vals_i], axis=1)
    ones_i = lax.bitcast_convert_type(_pad_edges(jnp.ones((NEDGE,), f32), 0.0),
                                      jnp.int32)
    deg_edges = jnp.stack([jnp.zeros_like(tail_p), head_p, ones_i], axis=1)

    rel2 = rel_emb.astype(f32).reshape(4 * (NREL - 1), 16)
    all2 = all_embed.astype(f32).reshape(4 * (NU + NENT), 16)
    icf2 = item_emb_cf.astype(f32).reshape(4 * NI, 16)

    ones_tbl = jnp.ones((4, 16), f32)
    seg_deg = _make_segsum(4, ACC_E, RPT_E, "deg", 0)
    deg_full = seg_deg(ones_tbl, deg_edges)
    deg8 = deg_full[:, 0, 0:8]

    e_res = jnp.concatenate([all_embed[NU:], jnp.zeros((ACC_E - NENT, EMB), f32)])
    u_res = jnp.concatenate([all_embed[:NU], jnp.zeros((ACC_E - NU, EMB), f32)])
    i_res = jnp.concatenate([item_emb_cf, jnp.zeros((ACC_I - NI, EMB), f32)])

    seg_kg_0 = _make_segsum(4 * (NU + NENT), ACC_E, RPT_E, "rel", NU)
    seg_ua_0 = _make_segsum(4 * (NU + NENT), ACC_E, RPT_E, "val", NU)
    seg_kg = _make_segsum(4 * ACC_E, ACC_E, RPT_E, "rel", 0)
    seg_ua = _make_segsum(4 * ACC_E, ACC_E, RPT_E, "val", 0)
    seg_ucf_0 = _make_segsum(4 * NI, ACC_E, RPT_E, "val", 0)
    seg_ucf = _make_segsum(4 * ACC_I, ACC_E, RPT_E, "val", 0)
    seg_icf = _make_segsum(4 * ACC_E, ACC_I, RPT_I, "val", 0)

    ent2 = all2
    icf_t = icf2
    for hop in range(HOPS):
        if hop == 0:
            agg = seg_kg_0(ent2, kg_edges, rel2)
            uagg = seg_ua_0(ent2, ua_edges)
            ucf = seg_ucf_0(icf_t, ua_edges)
        else:
            agg = seg_kg(ent2, kg_edges, rel2)
            uagg = seg_ua(ent2, ua_edges)
            ucf = seg_ucf(icf_t, ua_edges)
        icf_new = seg_icf(ucf.reshape(4 * ACC_E, 16), icf_edges)

        ent_next, e_res = _div_norm_res(agg.reshape(ACC_E, EMB), deg8, e_res)
        u_res = _norm_res(uagg.reshape(ACC_E, EMB), u_res)
        i_res = _norm_res(icf_new.reshape(ACC_I, EMB), i_res)

        ent2 = ent_next.reshape(4 * ACC_E, 16)
        icf_t = icf_new.reshape(4 * ACC_I, 16)

    bg = _make_batch_gather()
    neg_flat = neg_item.reshape(-1).astype(jnp.int32)
    oue, ope, opc, one, onc = bg(
        u_res.reshape(2 * ACC_E, 32),
        e_res.reshape(2 * ACC_E, 32),
        i_res.reshape(2 * ACC_I, 32),
        user.astype(jnp.int32).reshape(-1, CH),
        pos_item.astype(jnp.int32).reshape(-1, CH),
        neg_flat.reshape(-1, CH))

    tot, su, sp, sne, spc, snc = _loss_partials(
        oue.reshape(BATCH, EMB),
        ope.reshape(BATCH, EMB),
        opc.reshape(BATCH, EMB),
        one.reshape(BATCH, NEG * EMB),
        onc.reshape(BATCH, NEG * EMB))
    loss1 = tot[0, 0] / BATCH
    reg = DECAY * (su + sp + sne + spc + snc)[0, 0] / (2.0 * BATCH)

    et_n = tri_head.shape[0]
    gran = NC * NS * CH * 2
    tpad = max((et_n + gran - 1) // gran, 1) * gran
    th_p = jnp.concatenate(
        [tri_head.astype(jnp.int32),
         jnp.zeros((tpad - et_n,), jnp.int32)]).reshape(-1, CH)
    tt_p = jnp.concatenate(
        [tri_tail.astype(jnp.int32),
         jnp.zeros((tpad - et_n,), jnp.int32)]).reshape(-1, CH)
    ang = _make_angle(tpad)
    dots = ang(all_embed.astype(f32), th_p, tt_p)

    sc = ANGLE_DROP * ANGLE_DROP
    hh = dots[:et_n, 0] * sc
    tt = dots[:et_n, 1] * sc
    ht = dots[:et_n, 2] * sc
    eps = 1e-6
    nu_ = jnp.sqrt(hh)
    edist = jnp.sqrt(jnp.maximum(hh + tt - 2.0 * ht, 0.0))
    num = ht * (1.0 + hh) - hh * (1.0 + tt)
    denom = nu_ * edist * jnp.sqrt(jnp.clip(1.0 + tt * hh - 2.0 * ht, eps)) + eps
    angle = jnp.arccos(jnp.clip(num / denom, -1.0 + eps, 1.0 - eps))
    sqnu = jnp.clip(hh, 0.0, 1.0 - eps)
    half_ap = jnp.arcsin(jnp.clip(0.1 * (1.0 - sqnu) / jnp.sqrt(sqnu + eps),
                                  -1.0 + eps, 1.0 - eps))
    loss2 = ANGLE_W * jnp.sum(jnp.maximum(angle - half_ap, 0.0)) / et_n

    return loss1 + reg + loss2
